# Initial kernel scaffold; baseline (speedup 1.0000x reference)
import jax
import jax.numpy as jnp
from jax.experimental import pallas as pl


def kernel(edge_index, x, src_nodes, dst_nodes, user_emb, book_emb, W1, b1, g1, be1, W2, b2, g2, be2):
    raise NotImplementedError("write your pallas kernel here")


# speedup vs baseline: 25.5090x; 25.5090x over previous
import functools

import jax
import jax.numpy as jnp
from jax import lax
from jax.experimental import pallas as pl
from jax.experimental.pallas import tpu as pltpu
from jax.experimental.pallas import tpu_sc as plsc

NN = 100000
EMB = 32
HALF = 16
E = 1600000
NC, NS = 2, 16

CHUNK = 1024
KJ = CHUNK // 128
EPT = 100352
NCH = EPT // CHUNK
EPAD = EPT * NS
ROWS_PT = 6256
ROWS_LAST = NN - 15 * ROWS_PT
TRASH = 96
ACC_ROWS = NN + TRASH

DCHUNK = 1024
KJD = DCHUNK // 128
EPT_D = EPAD // (NC * NS)
NCH_D = EPT_D // DCHUNK
DEG_ROWS = NN + 96
DROWS_PT = DEG_ROWS // NS

GBLK = 10000
GRID = NN // GBLK

NP = 16384
PPT = NP // (NC * NS)
PJ = PPT // 128

_MESH = plsc.VectorSubcoreMesh(
    core_axis_name="c", subcore_axis_name="s", num_cores=NC, num_subcores=NS)
_SC_PARAMS = pltpu.CompilerParams(
    use_tc_tiling_on_sc=False, needs_layout_passes=False)


@functools.partial(
    pl.kernel,
    out_type=(jax.ShapeDtypeStruct((DEG_ROWS,), jnp.float32),
              jax.ShapeDtypeStruct((DEG_ROWS,), jnp.float32)),
    mesh=_MESH,
    compiler_params=_SC_PARAMS,
    scratch_types=[
        pltpu.VMEM_SHARED((DEG_ROWS,), jnp.float32),
        pltpu.VMEM((KJD, 128), jnp.int32),
        pltpu.VMEM((DROWS_PT,), jnp.float32),
        pltpu.VMEM((128,), jnp.float32),
        pltpu.SemaphoreType.DMA,
    ],
)
def _deg_kernel(dst_hbm, d0_hbm, d1_hbm, dacc, idx_d, zbuf, ones_v, sem):
    cid = lax.axis_index("c")
    sid = lax.axis_index("s")
    r0 = sid * DROWS_PT

    def zb(i, carry):
        zbuf[pl.ds(i * 16, 16)] = jnp.zeros((16,), jnp.float32)
        return carry

    lax.fori_loop(0, DROWS_PT // 16, zb, 0)

    def ob(i, carry):
        ones_v[pl.ds(i * 16, 16)] = jnp.ones((16,), jnp.float32)
        return carry

    lax.fori_loop(0, 8, ob, 0)
    pltpu.sync_copy(zbuf, dacc.at[pl.ds(r0, DROWS_PT)])
    plsc.subcore_barrier()
    wid = cid * NS + sid
    base128 = wid * (EPT_D // 128)

    def body(ch, carry):
        off = base128 + ch * KJD
        pltpu.async_copy(dst_hbm.at[pl.ds(off, KJD)], idx_d, sem).wait()
        for j in range(KJD):
            pltpu.sync_copy(ones_v, dacc.at[idx_d.at[j]], add=True)
        return carry

    lax.fori_loop(0, NCH_D, body, 0)
    plsc.subcore_barrier()
    pltpu.sync_copy(dacc.at[pl.ds(r0, DROWS_PT)], zbuf)

    @pl.when(cid == 0)
    def _():
        pltpu.sync_copy(zbuf, d0_hbm.at[pl.ds(r0, DROWS_PT)])

    @pl.when(cid == 1)
    def _():
        pltpu.sync_copy(zbuf, d1_hbm.at[pl.ds(r0, DROWS_PT)])


@functools.partial(
    pl.kernel,
    out_type=(jax.ShapeDtypeStruct((NN, HALF), jnp.float32),
              jax.ShapeDtypeStruct((NN, HALF), jnp.float32)),
    mesh=_MESH,
    compiler_params=_SC_PARAMS,
    scratch_types=[
        pltpu.VMEM_SHARED((ACC_ROWS, HALF), jnp.float32),
        pltpu.VMEM((KJ, 128), jnp.int32),
        pltpu.VMEM((KJ, 128), jnp.int32),
        pltpu.VMEM((CHUNK, HALF), jnp.float32),
        pltpu.SemaphoreType.DMA,
        pltpu.SemaphoreType.DMA,
    ],
)
def _mp_kernel(q0_hbm, q1_hbm, src_hbm, dst_hbm, a0_hbm, a1_hbm,
               acc, idx_s, idx_d, rows, gsem, isem):
    cid = lax.axis_index("c")
    sid = lax.axis_index("s")
    r0 = sid * ROWS_PT
    ch_full = (1024,) * 6 + (ROWS_PT - 6 * 1024,)
    ch_last = (1024,) * 6 + (ROWS_LAST - 6 * 1024,)

    def stage(hbm_ref, chunks, to_spmem):
        off = 0
        for cnt in chunks:
            hslice = hbm_ref.at[pl.ds(r0 + off, cnt)]
            vslice = rows.at[pl.ds(0, cnt)]
            aslice = acc.at[pl.ds(r0 + off, cnt)]
            if to_spmem:
                pltpu.sync_copy(hslice, vslice)
                pltpu.sync_copy(vslice, aslice)
            else:
                pltpu.sync_copy(aslice, vslice)
                pltpu.sync_copy(vslice, hslice)
            off += cnt

    def stage_in(hbm_ref):
        @pl.when(sid != 15)
        def _():
            stage(hbm_ref, ch_full, True)

        @pl.when(sid == 15)
        def _():
            stage(hbm_ref, ch_last, True)

            def zb(i, carry):
                rows[i, pl.ds(0, HALF)] = jnp.zeros((HALF,), jnp.float32)
                return carry

            lax.fori_loop(0, TRASH, zb, 0)
            pltpu.sync_copy(rows.at[pl.ds(0, TRASH)], acc.at[pl.ds(NN, TRASH)])

    @pl.when(cid == 0)
    def _():
        stage_in(q0_hbm)

    @pl.when(cid == 1)
    def _():
        stage_in(q1_hbm)

    plsc.subcore_barrier()
    base128 = sid * (EPT // 128)

    def run(q_hbm):
        def body(ch, carry):
            off = base128 + ch * KJ
            c1 = pltpu.async_copy(src_hbm.at[pl.ds(off, KJ)], idx_s, isem)
            c2 = pltpu.async_copy(dst_hbm.at[pl.ds(off, KJ)], idx_d, isem)
            c1.wait()
            c2.wait()
            cps = [pltpu.async_copy(q_hbm.at[idx_s.at[j]],
                                    rows.at[pl.ds(j * 128, 128)], gsem)
                   for j in range(KJ)]
            for cp in cps:
                cp.wait()
            for j in range(KJ):
                pltpu.sync_copy(rows.at[pl.ds(j * 128, 128)],
                                acc.at[idx_d.at[j]], add=True)
            return carry

        lax.fori_loop(0, NCH, body, 0)

    @pl.when(cid == 0)
    def _():
        run(q0_hbm)

    @pl.when(cid == 1)
    def _():
        run(q1_hbm)

    plsc.subcore_barrier()

    def stage_out(hbm_ref):
        @pl.when(sid != 15)
        def _():
            stage(hbm_ref, ch_full, False)

        @pl.when(sid == 15)
        def _():
            stage(hbm_ref, ch_last, False)

    @pl.when(cid == 0)
    def _():
        stage_out(a0_hbm)

    @pl.when(cid == 1)
    def _():
        stage_out(a1_hbm)


@functools.partial(
    pl.kernel,
    out_type=jax.ShapeDtypeStruct((NP,), jnp.float32),
    mesh=_MESH,
    compiler_params=_SC_PARAMS,
    scratch_types=[
        pltpu.VMEM((PJ, 128), jnp.int32),
        pltpu.VMEM((PJ, 128), jnp.int32),
        pltpu.VMEM((128, EMB), jnp.float32),
        pltpu.VMEM((128, EMB), jnp.float32),
        pltpu.VMEM((8, EMB), jnp.float32),
        pltpu.VMEM((PPT,), jnp.float32),
        pltpu.SemaphoreType.DMA,
    ],
)
def _gd_kernel(h_hbm, ac_hbm, srcn_hbm, dstn_hbm, r_hbm,
               idx_a, idx_b, bufa, bufb, ac_v, res, sem):
    cid = lax.axis_index("c")
    sid = lax.axis_index("s")
    wid = cid * NS + sid
    pltpu.sync_copy(ac_hbm, ac_v)
    pltpu.sync_copy(srcn_hbm.at[pl.ds(wid * PJ, PJ)], idx_a)
    pltpu.sync_copy(dstn_hbm.at[pl.ds(wid * PJ, PJ)], idx_b)
    lane = lax.iota(jnp.int32, 16)
    a_lo = ac_v[0, pl.ds(0, 16)]
    a_hi = ac_v[0, pl.ds(16, 16)]
    c_lo = ac_v[1, pl.ds(0, 16)]
    c_hi = ac_v[1, pl.ds(16, 16)]

    def jbody(j, carry):
        pltpu.async_copy(h_hbm.at[idx_a.at[j]], bufa, sem).wait()
        pltpu.async_copy(h_hbm.at[idx_b.at[j]], bufb, sem).wait()

        def rbody(r, carry2):
            bufa[r, pl.ds(0, 16)] = jnp.maximum(
                bufa[r, pl.ds(0, 16)] * a_lo + c_lo, 0.0)
            bufa[r, pl.ds(16, 16)] = jnp.maximum(
                bufa[r, pl.ds(16, 16)] * a_hi + c_hi, 0.0)
            bufb[r, pl.ds(0, 16)] = jnp.maximum(
                bufb[r, pl.ds(0, 16)] * a_lo + c_lo, 0.0)
            bufb[r, pl.ds(16, 16)] = jnp.maximum(
                bufb[r, pl.ds(16, 16)] * a_hi + c_hi, 0.0)
            return carry2

        lax.fori_loop(0, 128, rbody, 0)

        def gbody(g, carry2):
            rowi = g * 16 + lane

            def cbody(cc, acc16):
                col = jnp.zeros((16,), jnp.int32) + cc
                av = plsc.load_gather(bufa, [rowi, col])
                bv = plsc.load_gather(bufb, [rowi, col])
                return acc16 + av * bv

            acc16 = lax.fori_loop(0, EMB, cbody, jnp.zeros((16,), jnp.float32))
            res[pl.ds(j * 128 + g * 16, 16)] = acc16
            return carry2

        lax.fori_loop(0, 8, gbody, 0)
        return carry

    lax.fori_loop(0, PJ, jbody, 0)
    pltpu.sync_copy(res, r_hbm.at[pl.ds(wid * PPT, PPT)])


def _tc1_body(h_ref, dp0_ref, dp1_ref, w_ref, q0_ref, q1_ref):
    deg = dp0_ref[...] + dp1_ref[...] + 1.0
    dinv = lax.rsqrt(deg)
    q = jnp.dot(h_ref[...], w_ref[...],
                preferred_element_type=jnp.float32) * dinv
    q0_ref[...] = q[:, :HALF]
    q1_ref[...] = q[:, HALF:]


def _tc2_body(s0_ref, s1_ref, dp0_ref, dp1_ref, b_ref, g_ref, be_ref,
              out_ref, ac_ref, accum):
    i = pl.program_id(0)
    deg = dp0_ref[...] + dp1_ref[...] + 1.0
    dinv = lax.rsqrt(deg)
    s = jnp.concatenate([s0_ref[...], s1_ref[...]], axis=1)
    out = s * dinv + b_ref[...]
    out_ref[...] = out

    @pl.when(i == 0)
    def _():
        accum[...] = jnp.zeros_like(accum)

    accum[0:1, :] += jnp.sum(out, axis=0, keepdims=True)
    accum[1:2, :] += jnp.sum(out * out, axis=0, keepdims=True)

    @pl.when(i == GRID - 1)
    def _():
        mu = accum[0:1, :] / NN
        var = accum[1:2, :] / NN - mu * mu
        a = g_ref[...] * lax.rsqrt(var + 1e-5)
        c = be_ref[...] - a * mu
        ac_ref[0:1, :] = a
        ac_ref[1:2, :] = c
        ac_ref[2:8, :] = jnp.zeros((6, EMB), jnp.float32)


def _tc3_body(out_ref, ac_ref, dp0_ref, dp1_ref, w_ref, q0_ref, q1_ref):
    deg = dp0_ref[...] + dp1_ref[...] + 1.0
    dinv = lax.rsqrt(deg)
    h = jnp.maximum(out_ref[...] * ac_ref[0:1, :] + ac_ref[1:2, :], 0.0)
    q = jnp.dot(h, w_ref[...], preferred_element_type=jnp.float32) * dinv
    q0_ref[...] = q[:, :HALF]
    q1_ref[...] = q[:, HALF:]


def _row_spec(cols):
    return pl.BlockSpec((GBLK, cols), lambda i: (i, 0))


def _full_spec(rows, cols):
    return pl.BlockSpec((rows, cols), lambda i: (0, 0))


_tc1 = pl.pallas_call(
    _tc1_body,
    grid=(GRID,),
    in_specs=[_row_spec(EMB), _row_spec(1), _row_spec(1), _full_spec(EMB, EMB)],
    out_specs=[_row_spec(HALF), _row_spec(HALF)],
    out_shape=[jax.ShapeDtypeStruct((NN, HALF), jnp.float32)] * 2,
)

_tc2 = pl.pallas_call(
    _tc2_body,
    grid=(GRID,),
    in_specs=[_row_spec(HALF), _row_spec(HALF), _row_spec(1), _row_spec(1),
              _full_spec(1, EMB), _full_spec(1, EMB), _full_spec(1, EMB)],
    out_specs=[_row_spec(EMB), _full_spec(8, EMB)],
    out_shape=[jax.ShapeDtypeStruct((NN, EMB), jnp.float32),
               jax.ShapeDtypeStruct((8, EMB), jnp.float32)],
    scratch_shapes=[pltpu.VMEM((8, EMB), jnp.float32)],
)

_tc3 = pl.pallas_call(
    _tc3_body,
    grid=(GRID,),
    in_specs=[_row_spec(EMB), _full_spec(8, EMB), _row_spec(1), _row_spec(1),
              _full_spec(EMB, EMB)],
    out_specs=[_row_spec(HALF), _row_spec(HALF)],
    out_shape=[jax.ShapeDtypeStruct((NN, HALF), jnp.float32)] * 2,
)


def kernel(edge_index, x, src_nodes, dst_nodes, user_emb, book_emb,
           W1, b1, g1, be1, W2, b2, g2, be2):
    h = jnp.concatenate([user_emb, book_emb], axis=0)
    pad = EPAD - E
    srcp = jnp.concatenate(
        [edge_index[0], jnp.zeros((pad,), jnp.int32)]).reshape(EPAD // 128, 128)
    dstp = jnp.concatenate(
        [edge_index[1], jnp.full((pad,), NN, jnp.int32)]).reshape(EPAD // 128, 128)
    d0, d1 = _deg_kernel(dstp)
    dp0 = d0[:NN].reshape(NN, 1)
    dp1 = d1[:NN].reshape(NN, 1)

    q1a, q1b = _tc1(h, dp0, dp1, W1)
    s1a, s1b = _mp_kernel(q1a, q1b, srcp, dstp)
    out1, ac1 = _tc2(s1a, s1b, dp0, dp1, b1.reshape(1, EMB),
                     g1.reshape(1, EMB), be1.reshape(1, EMB))
    q2a, q2b = _tc3(out1, ac1, dp0, dp1, W2)
    s2a, s2b = _mp_kernel(q2a, q2b, srcp, dstp)
    out2, ac2 = _tc2(s2a, s2b, dp0, dp1, b2.reshape(1, EMB),
                     g2.reshape(1, EMB), be2.reshape(1, EMB))
    ratings = _gd_kernel(out2, ac2, src_nodes.reshape(128, 128),
                         dst_nodes.reshape(128, 128))
    return ratings


# speedup vs baseline: 26.9049x; 1.0547x over previous
import functools

import jax
import jax.numpy as jnp
from jax import lax
from jax.experimental import pallas as pl
from jax.experimental.pallas import tpu as pltpu
from jax.experimental.pallas import tpu_sc as plsc

NN = 100000
EMB = 32
HALF = 16
E = 1600000
NC, NS = 2, 16

CHUNK = 512
KJ = CHUNK // 128
EPT = 100352
NCH = EPT // CHUNK
EPAD = EPT * NS
ROWS_PT = 6256
ROWS_LAST = NN - 15 * ROWS_PT
TRASH = 96
ACC_ROWS = NN + TRASH

DCHUNK = 1024
KJD = DCHUNK // 128
EPT_D = EPAD // (NC * NS)
NCH_D = EPT_D // DCHUNK
DEG_ROWS = NN + 96
DROWS_PT = DEG_ROWS // NS

GBLK = 10000
GRID = NN // GBLK

NP = 16384
PPT = NP // (NC * NS)
PJ = PPT // 128

_MESH = plsc.VectorSubcoreMesh(
    core_axis_name="c", subcore_axis_name="s", num_cores=NC, num_subcores=NS)
_SC_PARAMS = pltpu.CompilerParams(
    use_tc_tiling_on_sc=False, needs_layout_passes=False)


@functools.partial(
    pl.kernel,
    out_type=(jax.ShapeDtypeStruct((DEG_ROWS,), jnp.float32),
              jax.ShapeDtypeStruct((DEG_ROWS,), jnp.float32)),
    mesh=_MESH,
    compiler_params=_SC_PARAMS,
    scratch_types=[
        pltpu.VMEM_SHARED((DEG_ROWS,), jnp.float32),
        pltpu.VMEM((KJD, 128), jnp.int32),
        pltpu.VMEM((DROWS_PT,), jnp.float32),
        pltpu.VMEM((128,), jnp.float32),
        pltpu.SemaphoreType.DMA,
    ],
)
def _deg_kernel(dst_hbm, d0_hbm, d1_hbm, dacc, idx_d, zbuf, ones_v, sem):
    cid = lax.axis_index("c")
    sid = lax.axis_index("s")
    r0 = sid * DROWS_PT

    def zb(i, carry):
        zbuf[pl.ds(i * 16, 16)] = jnp.zeros((16,), jnp.float32)
        return carry

    lax.fori_loop(0, DROWS_PT // 16, zb, 0)

    def ob(i, carry):
        ones_v[pl.ds(i * 16, 16)] = jnp.ones((16,), jnp.float32)
        return carry

    lax.fori_loop(0, 8, ob, 0)
    pltpu.sync_copy(zbuf, dacc.at[pl.ds(r0, DROWS_PT)])
    plsc.subcore_barrier()
    wid = cid * NS + sid
    base128 = wid * (EPT_D // 128)

    def body(ch, carry):
        off = base128 + ch * KJD
        pltpu.async_copy(dst_hbm.at[pl.ds(off, KJD)], idx_d, sem).wait()
        for j in range(KJD):
            pltpu.sync_copy(ones_v, dacc.at[idx_d.at[j]], add=True)
        return carry

    lax.fori_loop(0, NCH_D, body, 0)
    plsc.subcore_barrier()
    pltpu.sync_copy(dacc.at[pl.ds(r0, DROWS_PT)], zbuf)

    @pl.when(cid == 0)
    def _():
        pltpu.sync_copy(zbuf, d0_hbm.at[pl.ds(r0, DROWS_PT)])

    @pl.when(cid == 1)
    def _():
        pltpu.sync_copy(zbuf, d1_hbm.at[pl.ds(r0, DROWS_PT)])


@functools.partial(
    pl.kernel,
    out_type=(jax.ShapeDtypeStruct((NN, HALF), jnp.float32),
              jax.ShapeDtypeStruct((NN, HALF), jnp.float32)),
    mesh=_MESH,
    compiler_params=_SC_PARAMS,
    scratch_types=[
        pltpu.VMEM_SHARED((ACC_ROWS, HALF), jnp.float32),
        pltpu.VMEM((2 * KJ, 128), jnp.int32),
        pltpu.VMEM((2 * KJ, 128), jnp.int32),
        pltpu.VMEM((CHUNK, HALF), jnp.float32),
        pltpu.VMEM((CHUNK, HALF), jnp.float32),
        pltpu.SemaphoreType.DMA,
        pltpu.SemaphoreType.DMA,
        pltpu.SemaphoreType.DMA,
    ],
)
def _mp_kernel(q0_hbm, q1_hbm, src_hbm, dst_hbm, a0_hbm, a1_hbm,
               acc, idx_s, idx_d, rows0, rows1, gsem0, gsem1, isem):
    cid = lax.axis_index("c")
    sid = lax.axis_index("s")
    r0 = sid * ROWS_PT
    ch_full = (512,) * 12 + (ROWS_PT - 12 * 512,)
    ch_last = (512,) * 12 + (ROWS_LAST - 12 * 512,)

    def stage(hbm_ref, chunks, to_spmem):
        off = 0
        for cnt in chunks:
            hslice = hbm_ref.at[pl.ds(r0 + off, cnt)]
            vslice = rows0.at[pl.ds(0, cnt)]
            aslice = acc.at[pl.ds(r0 + off, cnt)]
            if to_spmem:
                pltpu.sync_copy(hslice, vslice)
                pltpu.sync_copy(vslice, aslice)
            else:
                pltpu.sync_copy(aslice, vslice)
                pltpu.sync_copy(vslice, hslice)
            off += cnt

    def stage_in(hbm_ref):
        @pl.when(sid != 15)
        def _():
            stage(hbm_ref, ch_full, True)

        @pl.when(sid == 15)
        def _():
            stage(hbm_ref, ch_last, True)

            def zb(i, carry):
                rows0[i, pl.ds(0, HALF)] = jnp.zeros((HALF,), jnp.float32)
                return carry

            lax.fori_loop(0, TRASH, zb, 0)
            pltpu.sync_copy(rows0.at[pl.ds(0, TRASH)], acc.at[pl.ds(NN, TRASH)])

    @pl.when(cid == 0)
    def _():
        stage_in(q0_hbm)

    @pl.when(cid == 1)
    def _():
        stage_in(q1_hbm)

    plsc.subcore_barrier()
    base128 = sid * (EPT // 128)

    def run(q_hbm):
        def fire(rows_buf, sem, j0):
            for j in range(KJ):
                pltpu.async_copy(q_hbm.at[idx_s.at[j0 + j]],
                                 rows_buf.at[pl.ds(j * 128, 128)], sem)

        def drain(rows_buf, sem):
            pltpu.make_async_copy(q_hbm.at[pl.ds(0, CHUNK)], rows_buf,
                                  sem).wait()

        def scatter(rows_buf, j0):
            for j in range(KJ):
                pltpu.sync_copy(rows_buf.at[pl.ds(j * 128, 128)],
                                acc.at[idx_d.at[j0 + j]], add=True)

        def body(chp, carry):
            off = base128 + chp * (2 * KJ)
            c1 = pltpu.async_copy(src_hbm.at[pl.ds(off, 2 * KJ)], idx_s, isem)
            c2 = pltpu.async_copy(dst_hbm.at[pl.ds(off, 2 * KJ)], idx_d, isem)
            c1.wait()
            c2.wait()
            fire(rows0, gsem0, 0)
            fire(rows1, gsem1, KJ)
            drain(rows0, gsem0)
            scatter(rows0, 0)
            drain(rows1, gsem1)
            scatter(rows1, KJ)
            return carry

        lax.fori_loop(0, NCH // 2, body, 0)

    @pl.when(cid == 0)
    def _():
        run(q0_hbm)

    @pl.when(cid == 1)
    def _():
        run(q1_hbm)

    plsc.subcore_barrier()

    def stage_out(hbm_ref):
        @pl.when(sid != 15)
        def _():
            stage(hbm_ref, ch_full, False)

        @pl.when(sid == 15)
        def _():
            stage(hbm_ref, ch_last, False)

    @pl.when(cid == 0)
    def _():
        stage_out(a0_hbm)

    @pl.when(cid == 1)
    def _():
        stage_out(a1_hbm)


@functools.partial(
    pl.kernel,
    out_type=jax.ShapeDtypeStruct((NP,), jnp.float32),
    mesh=_MESH,
    compiler_params=_SC_PARAMS,
    scratch_types=[
        pltpu.VMEM((PJ, 128), jnp.int32),
        pltpu.VMEM((PJ, 128), jnp.int32),
        pltpu.VMEM((128, EMB), jnp.float32),
        pltpu.VMEM((128, EMB), jnp.float32),
        pltpu.VMEM((8, EMB), jnp.float32),
        pltpu.VMEM((PPT,), jnp.float32),
        pltpu.SemaphoreType.DMA,
    ],
)
def _gd_kernel(h_hbm, ac_hbm, srcn_hbm, dstn_hbm, r_hbm,
               idx_a, idx_b, bufa, bufb, ac_v, res, sem):
    cid = lax.axis_index("c")
    sid = lax.axis_index("s")
    wid = cid * NS + sid
    pltpu.sync_copy(ac_hbm, ac_v)
    pltpu.sync_copy(srcn_hbm.at[pl.ds(wid * PJ, PJ)], idx_a)
    pltpu.sync_copy(dstn_hbm.at[pl.ds(wid * PJ, PJ)], idx_b)
    lane = lax.iota(jnp.int32, 16)
    a_lo = ac_v[0, pl.ds(0, 16)]
    a_hi = ac_v[0, pl.ds(16, 16)]
    c_lo = ac_v[1, pl.ds(0, 16)]
    c_hi = ac_v[1, pl.ds(16, 16)]

    def jbody(j, carry):
        pltpu.async_copy(h_hbm.at[idx_a.at[j]], bufa, sem).wait()
        pltpu.async_copy(h_hbm.at[idx_b.at[j]], bufb, sem).wait()

        def rbody(r, carry2):
            bufa[r, pl.ds(0, 16)] = jnp.maximum(
                bufa[r, pl.ds(0, 16)] * a_lo + c_lo, 0.0)
            bufa[r, pl.ds(16, 16)] = jnp.maximum(
                bufa[r, pl.ds(16, 16)] * a_hi + c_hi, 0.0)
            bufb[r, pl.ds(0, 16)] = jnp.maximum(
                bufb[r, pl.ds(0, 16)] * a_lo + c_lo, 0.0)
            bufb[r, pl.ds(16, 16)] = jnp.maximum(
                bufb[r, pl.ds(16, 16)] * a_hi + c_hi, 0.0)
            return carry2

        lax.fori_loop(0, 128, rbody, 0)

        def gbody(g, carry2):
            rowi = g * 16 + lane

            def cbody(cc, acc16):
                col = jnp.zeros((16,), jnp.int32) + cc
                av = plsc.load_gather(bufa, [rowi, col])
                bv = plsc.load_gather(bufb, [rowi, col])
                return acc16 + av * bv

            acc16 = lax.fori_loop(0, EMB, cbody, jnp.zeros((16,), jnp.float32))
            res[pl.ds(j * 128 + g * 16, 16)] = acc16
            return carry2

        lax.fori_loop(0, 8, gbody, 0)
        return carry

    lax.fori_loop(0, PJ, jbody, 0)
    pltpu.sync_copy(res, r_hbm.at[pl.ds(wid * PPT, PPT)])


def _tc1_body(h_ref, dp0_ref, dp1_ref, w_ref, q0_ref, q1_ref):
    deg = dp0_ref[...] + dp1_ref[...] + 1.0
    dinv = lax.rsqrt(deg)
    q = jnp.dot(h_ref[...], w_ref[...],
                preferred_element_type=jnp.float32) * dinv
    q0_ref[...] = q[:, :HALF]
    q1_ref[...] = q[:, HALF:]


def _tc2_body(s0_ref, s1_ref, dp0_ref, dp1_ref, b_ref, g_ref, be_ref,
              out_ref, ac_ref, accum):
    i = pl.program_id(0)
    deg = dp0_ref[...] + dp1_ref[...] + 1.0
    dinv = lax.rsqrt(deg)
    s = jnp.concatenate([s0_ref[...], s1_ref[...]], axis=1)
    out = s * dinv + b_ref[...]
    out_ref[...] = out

    @pl.when(i == 0)
    def _():
        accum[...] = jnp.zeros_like(accum)

    accum[0:1, :] += jnp.sum(out, axis=0, keepdims=True)
    accum[1:2, :] += jnp.sum(out * out, axis=0, keepdims=True)

    @pl.when(i == GRID - 1)
    def _():
        mu = accum[0:1, :] / NN
        var = accum[1:2, :] / NN - mu * mu
        a = g_ref[...] * lax.rsqrt(var + 1e-5)
        c = be_ref[...] - a * mu
        ac_ref[0:1, :] = a
        ac_ref[1:2, :] = c
        ac_ref[2:8, :] = jnp.zeros((6, EMB), jnp.float32)


def _tc3_body(out_ref, ac_ref, dp0_ref, dp1_ref, w_ref, q0_ref, q1_ref):
    deg = dp0_ref[...] + dp1_ref[...] + 1.0
    dinv = lax.rsqrt(deg)
    h = jnp.maximum(out_ref[...] * ac_ref[0:1, :] + ac_ref[1:2, :], 0.0)
    q = jnp.dot(h, w_ref[...], preferred_element_type=jnp.float32) * dinv
    q0_ref[...] = q[:, :HALF]
    q1_ref[...] = q[:, HALF:]


def _row_spec(cols):
    return pl.BlockSpec((GBLK, cols), lambda i: (i, 0))


def _full_spec(rows, cols):
    return pl.BlockSpec((rows, cols), lambda i: (0, 0))


_tc1 = pl.pallas_call(
    _tc1_body,
    grid=(GRID,),
    in_specs=[_row_spec(EMB), _row_spec(1), _row_spec(1), _full_spec(EMB, EMB)],
    out_specs=[_row_spec(HALF), _row_spec(HALF)],
    out_shape=[jax.ShapeDtypeStruct((NN, HALF), jnp.float32)] * 2,
)

_tc2 = pl.pallas_call(
    _tc2_body,
    grid=(GRID,),
    in_specs=[_row_spec(HALF), _row_spec(HALF), _row_spec(1), _row_spec(1),
              _full_spec(1, EMB), _full_spec(1, EMB), _full_spec(1, EMB)],
    out_specs=[_row_spec(EMB), _full_spec(8, EMB)],
    out_shape=[jax.ShapeDtypeStruct((NN, EMB), jnp.float32),
               jax.ShapeDtypeStruct((8, EMB), jnp.float32)],
    scratch_shapes=[pltpu.VMEM((8, EMB), jnp.float32)],
)

_tc3 = pl.pallas_call(
    _tc3_body,
    grid=(GRID,),
    in_specs=[_row_spec(EMB), _full_spec(8, EMB), _row_spec(1), _row_spec(1),
              _full_spec(EMB, EMB)],
    out_specs=[_row_spec(HALF), _row_spec(HALF)],
    out_shape=[jax.ShapeDtypeStruct((NN, HALF), jnp.float32)] * 2,
)


def kernel(edge_index, x, src_nodes, dst_nodes, user_emb, book_emb,
           W1, b1, g1, be1, W2, b2, g2, be2):
    h = jnp.concatenate([user_emb, book_emb], axis=0)
    pad = EPAD - E
    srcp = jnp.concatenate(
        [edge_index[0], jnp.zeros((pad,), jnp.int32)]).reshape(EPAD // 128, 128)
    dstp = jnp.concatenate(
        [edge_index[1], jnp.full((pad,), NN, jnp.int32)]).reshape(EPAD // 128, 128)
    d0, d1 = _deg_kernel(dstp)
    dp0 = d0[:NN].reshape(NN, 1)
    dp1 = d1[:NN].reshape(NN, 1)

    q1a, q1b = _tc1(h, dp0, dp1, W1)
    s1a, s1b = _mp_kernel(q1a, q1b, srcp, dstp)
    out1, ac1 = _tc2(s1a, s1b, dp0, dp1, b1.reshape(1, EMB),
                     g1.reshape(1, EMB), be1.reshape(1, EMB))
    q2a, q2b = _tc3(out1, ac1, dp0, dp1, W2)
    s2a, s2b = _mp_kernel(q2a, q2b, srcp, dstp)
    out2, ac2 = _tc2(s2a, s2b, dp0, dp1, b2.reshape(1, EMB),
                     g2.reshape(1, EMB), be2.reshape(1, EMB))
    ratings = _gd_kernel(out2, ac2, src_nodes.reshape(128, 128),
                         dst_nodes.reshape(128, 128))
    return ratings


# speedup vs baseline: 28.3188x; 1.0526x over previous
import functools

import jax
import jax.numpy as jnp
from jax import lax
from jax.experimental import pallas as pl
from jax.experimental.pallas import tpu as pltpu
from jax.experimental.pallas import tpu_sc as plsc

NN = 100000
EMB = 32
HALF = 16
E = 1600000
NC, NS = 2, 16

CHUNK = 512
KJ = CHUNK // 128
EPT = 100352
NCH = EPT // CHUNK
EPAD = EPT * NS
ROWS_PT = 6256
ROWS_LAST = NN - 15 * ROWS_PT
TRASH = 96
ACC_ROWS = NN + TRASH

DCHUNK = 1024
KJD = DCHUNK // 128
EPT_D = EPAD // (NC * NS)
NCH_D = EPT_D // DCHUNK
DEG_ROWS = NN + 96
DROWS_PT = DEG_ROWS // NS

GBLK = 10000
GRID = NN // GBLK

NP = 16384
PPT = NP // (NC * NS)
PJ = PPT // 128

_MESH = plsc.VectorSubcoreMesh(
    core_axis_name="c", subcore_axis_name="s", num_cores=NC, num_subcores=NS)
_SC_PARAMS = pltpu.CompilerParams(
    use_tc_tiling_on_sc=False, needs_layout_passes=False)


@functools.partial(
    pl.kernel,
    out_type=(jax.ShapeDtypeStruct((DEG_ROWS,), jnp.float32),
              jax.ShapeDtypeStruct((DEG_ROWS,), jnp.float32)),
    mesh=_MESH,
    compiler_params=_SC_PARAMS,
    scratch_types=[
        pltpu.VMEM_SHARED((DEG_ROWS,), jnp.float32),
        pltpu.VMEM((KJD, 128), jnp.int32),
        pltpu.VMEM((DROWS_PT,), jnp.float32),
        pltpu.VMEM((128,), jnp.float32),
        pltpu.SemaphoreType.DMA,
    ],
)
def _deg_kernel(dst_hbm, d0_hbm, d1_hbm, dacc, idx_d, zbuf, ones_v, sem):
    cid = lax.axis_index("c")
    sid = lax.axis_index("s")
    r0 = sid * DROWS_PT

    def zb(i, carry):
        zbuf[pl.ds(i * 16, 16)] = jnp.zeros((16,), jnp.float32)
        return carry

    lax.fori_loop(0, DROWS_PT // 16, zb, 0)

    def ob(i, carry):
        ones_v[pl.ds(i * 16, 16)] = jnp.ones((16,), jnp.float32)
        return carry

    lax.fori_loop(0, 8, ob, 0)
    pltpu.sync_copy(zbuf, dacc.at[pl.ds(r0, DROWS_PT)])
    plsc.subcore_barrier()
    wid = cid * NS + sid
    base128 = wid * (EPT_D // 128)

    def body(ch, carry):
        off = base128 + ch * KJD
        pltpu.async_copy(dst_hbm.at[pl.ds(off, KJD)], idx_d, sem).wait()
        for j in range(KJD):
            pltpu.sync_copy(ones_v, dacc.at[idx_d.at[j]], add=True)
        return carry

    lax.fori_loop(0, NCH_D, body, 0)
    plsc.subcore_barrier()
    pltpu.sync_copy(dacc.at[pl.ds(r0, DROWS_PT)], zbuf)

    @pl.when(cid == 0)
    def _():
        pltpu.sync_copy(zbuf, d0_hbm.at[pl.ds(r0, DROWS_PT)])

    @pl.when(cid == 1)
    def _():
        pltpu.sync_copy(zbuf, d1_hbm.at[pl.ds(r0, DROWS_PT)])


@functools.partial(
    pl.kernel,
    out_type=(jax.ShapeDtypeStruct((NN, HALF), jnp.float32),
              jax.ShapeDtypeStruct((NN, HALF), jnp.float32)),
    mesh=_MESH,
    compiler_params=_SC_PARAMS,
    scratch_types=[
        pltpu.VMEM_SHARED((ACC_ROWS, HALF), jnp.float32),
        pltpu.VMEM((2 * CHUNK,), jnp.int32),
        pltpu.VMEM((2 * KJ, 128), jnp.int32),
        pltpu.VMEM((CHUNK, HALF), jnp.float32),
        pltpu.VMEM((CHUNK, HALF), jnp.float32),
        pltpu.SemaphoreType.DMA,
        pltpu.SemaphoreType.DMA,
        pltpu.SemaphoreType.DMA,
        pltpu.SemaphoreType.DMA,
        pltpu.SemaphoreType.DMA,
    ],
)
def _mp_kernel(q0_hbm, q1_hbm, src_hbm, dst_hbm, a0_hbm, a1_hbm,
               acc, idx_s, idx_d, rows0, rows1,
               gsem0, gsem1, ssem0, ssem1, isem):
    cid = lax.axis_index("c")
    sid = lax.axis_index("s")
    r0 = sid * ROWS_PT
    ch_full = (512,) * 12 + (ROWS_PT - 12 * 512,)
    ch_last = (512,) * 12 + (ROWS_LAST - 12 * 512,)

    def stage(hbm_ref, chunks, to_spmem):
        off = 0
        for cnt in chunks:
            hslice = hbm_ref.at[pl.ds(r0 + off, cnt)]
            vslice = rows0.at[pl.ds(0, cnt)]
            aslice = acc.at[pl.ds(r0 + off, cnt)]
            if to_spmem:
                pltpu.sync_copy(hslice, vslice)
                pltpu.sync_copy(vslice, aslice)
            else:
                pltpu.sync_copy(aslice, vslice)
                pltpu.sync_copy(vslice, hslice)
            off += cnt

    def stage_in(hbm_ref):
        @pl.when(sid != 15)
        def _():
            stage(hbm_ref, ch_full, True)

        @pl.when(sid == 15)
        def _():
            stage(hbm_ref, ch_last, True)

            def zb(i, carry):
                rows0[i, pl.ds(0, HALF)] = jnp.zeros((HALF,), jnp.float32)
                return carry

            lax.fori_loop(0, TRASH, zb, 0)
            pltpu.sync_copy(rows0.at[pl.ds(0, TRASH)], acc.at[pl.ds(NN, TRASH)])

    @pl.when(cid == 0)
    def _():
        stage_in(q0_hbm)

    @pl.when(cid == 1)
    def _():
        stage_in(q1_hbm)

    plsc.subcore_barrier()
    base128 = sid * (EPT // 128)

    def run(q_hbm):
        def fire_gather(rows_buf, sem, slot):
            pltpu.async_copy(q_hbm.at[idx_s.at[pl.ds(slot * CHUNK, CHUNK)]],
                             rows_buf, sem)

        def drain(sem):
            pltpu.make_async_copy(q_hbm.at[pl.ds(0, CHUNK)], rows0,
                                  sem).wait()

        def fire_scatter(rows_buf, sem, j0):
            for j in range(KJ):
                pltpu.async_copy(rows_buf.at[pl.ds(j * 128, 128)],
                                 acc.at[idx_d.at[j0 + j]], sem, add=True)

        def body(chp, carry):
            off = base128 + chp * (2 * KJ)
            c1 = pltpu.async_copy(
                src_hbm.at[pl.ds(off * 128, 2 * CHUNK)], idx_s, isem)
            c2 = pltpu.async_copy(dst_hbm.at[pl.ds(off, 2 * KJ)], idx_d, isem)
            c1.wait()
            c2.wait()
            fire_gather(rows0, gsem0, 0)
            fire_gather(rows1, gsem1, 1)
            drain(gsem0)
            fire_scatter(rows0, ssem0, 0)
            drain(gsem1)
            fire_scatter(rows1, ssem1, KJ)
            drain(ssem0)
            drain(ssem1)
            return carry

        lax.fori_loop(0, NCH // 2, body, 0)

    @pl.when(cid == 0)
    def _():
        run(q0_hbm)

    @pl.when(cid == 1)
    def _():
        run(q1_hbm)

    plsc.subcore_barrier()

    def stage_out(hbm_ref):
        @pl.when(sid != 15)
        def _():
            stage(hbm_ref, ch_full, False)

        @pl.when(sid == 15)
        def _():
            stage(hbm_ref, ch_last, False)

    @pl.when(cid == 0)
    def _():
        stage_out(a0_hbm)

    @pl.when(cid == 1)
    def _():
        stage_out(a1_hbm)


@functools.partial(
    pl.kernel,
    out_type=jax.ShapeDtypeStruct((NP,), jnp.float32),
    mesh=_MESH,
    compiler_params=_SC_PARAMS,
    scratch_types=[
        pltpu.VMEM((PJ, 128), jnp.int32),
        pltpu.VMEM((PJ, 128), jnp.int32),
        pltpu.VMEM((128, EMB), jnp.float32),
        pltpu.VMEM((128, EMB), jnp.float32),
        pltpu.VMEM((8, EMB), jnp.float32),
        pltpu.VMEM((PPT,), jnp.float32),
        pltpu.SemaphoreType.DMA,
    ],
)
def _gd_kernel(h_hbm, ac_hbm, srcn_hbm, dstn_hbm, r_hbm,
               idx_a, idx_b, bufa, bufb, ac_v, res, sem):
    cid = lax.axis_index("c")
    sid = lax.axis_index("s")
    wid = cid * NS + sid
    pltpu.sync_copy(ac_hbm, ac_v)
    pltpu.sync_copy(srcn_hbm.at[pl.ds(wid * PJ, PJ)], idx_a)
    pltpu.sync_copy(dstn_hbm.at[pl.ds(wid * PJ, PJ)], idx_b)
    lane = lax.iota(jnp.int32, 16)
    a_lo = ac_v[0, pl.ds(0, 16)]
    a_hi = ac_v[0, pl.ds(16, 16)]
    c_lo = ac_v[1, pl.ds(0, 16)]
    c_hi = ac_v[1, pl.ds(16, 16)]

    def jbody(j, carry):
        pltpu.async_copy(h_hbm.at[idx_a.at[j]], bufa, sem).wait()
        pltpu.async_copy(h_hbm.at[idx_b.at[j]], bufb, sem).wait()

        def rbody(r, carry2):
            bufa[r, pl.ds(0, 16)] = jnp.maximum(
                bufa[r, pl.ds(0, 16)] * a_lo + c_lo, 0.0)
            bufa[r, pl.ds(16, 16)] = jnp.maximum(
                bufa[r, pl.ds(16, 16)] * a_hi + c_hi, 0.0)
            bufb[r, pl.ds(0, 16)] = jnp.maximum(
                bufb[r, pl.ds(0, 16)] * a_lo + c_lo, 0.0)
            bufb[r, pl.ds(16, 16)] = jnp.maximum(
                bufb[r, pl.ds(16, 16)] * a_hi + c_hi, 0.0)
            return carry2

        lax.fori_loop(0, 128, rbody, 0)

        def gbody(g, carry2):
            rowi = g * 16 + lane

            def cbody(cc, acc16):
                col = jnp.zeros((16,), jnp.int32) + cc
                av = plsc.load_gather(bufa, [rowi, col])
                bv = plsc.load_gather(bufb, [rowi, col])
                return acc16 + av * bv

            acc16 = lax.fori_loop(0, EMB, cbody, jnp.zeros((16,), jnp.float32))
            res[pl.ds(j * 128 + g * 16, 16)] = acc16
            return carry2

        lax.fori_loop(0, 8, gbody, 0)
        return carry

    lax.fori_loop(0, PJ, jbody, 0)
    pltpu.sync_copy(res, r_hbm.at[pl.ds(wid * PPT, PPT)])


def _tc1_body(h_ref, dp0_ref, dp1_ref, w_ref, q0_ref, q1_ref):
    deg = dp0_ref[...] + dp1_ref[...] + 1.0
    dinv = lax.rsqrt(deg)
    q = jnp.dot(h_ref[...], w_ref[...],
                preferred_element_type=jnp.float32) * dinv
    q0_ref[...] = q[:, :HALF]
    q1_ref[...] = q[:, HALF:]


def _tc2_body(s0_ref, s1_ref, dp0_ref, dp1_ref, b_ref, g_ref, be_ref,
              out_ref, ac_ref, accum):
    i = pl.program_id(0)
    deg = dp0_ref[...] + dp1_ref[...] + 1.0
    dinv = lax.rsqrt(deg)
    s = jnp.concatenate([s0_ref[...], s1_ref[...]], axis=1)
    out = s * dinv + b_ref[...]
    out_ref[...] = out

    @pl.when(i == 0)
    def _():
        accum[...] = jnp.zeros_like(accum)

    accum[0:1, :] += jnp.sum(out, axis=0, keepdims=True)
    accum[1:2, :] += jnp.sum(out * out, axis=0, keepdims=True)

    @pl.when(i == GRID - 1)
    def _():
        mu = accum[0:1, :] / NN
        var = accum[1:2, :] / NN - mu * mu
        a = g_ref[...] * lax.rsqrt(var + 1e-5)
        c = be_ref[...] - a * mu
        ac_ref[0:1, :] = a
        ac_ref[1:2, :] = c
        ac_ref[2:8, :] = jnp.zeros((6, EMB), jnp.float32)


def _tc3_body(out_ref, ac_ref, dp0_ref, dp1_ref, w_ref, q0_ref, q1_ref):
    deg = dp0_ref[...] + dp1_ref[...] + 1.0
    dinv = lax.rsqrt(deg)
    h = jnp.maximum(out_ref[...] * ac_ref[0:1, :] + ac_ref[1:2, :], 0.0)
    q = jnp.dot(h, w_ref[...], preferred_element_type=jnp.float32) * dinv
    q0_ref[...] = q[:, :HALF]
    q1_ref[...] = q[:, HALF:]


def _row_spec(cols):
    return pl.BlockSpec((GBLK, cols), lambda i: (i, 0))


def _full_spec(rows, cols):
    return pl.BlockSpec((rows, cols), lambda i: (0, 0))


_tc1 = pl.pallas_call(
    _tc1_body,
    grid=(GRID,),
    in_specs=[_row_spec(EMB), _row_spec(1), _row_spec(1), _full_spec(EMB, EMB)],
    out_specs=[_row_spec(HALF), _row_spec(HALF)],
    out_shape=[jax.ShapeDtypeStruct((NN, HALF), jnp.float32)] * 2,
)

_tc2 = pl.pallas_call(
    _tc2_body,
    grid=(GRID,),
    in_specs=[_row_spec(HALF), _row_spec(HALF), _row_spec(1), _row_spec(1),
              _full_spec(1, EMB), _full_spec(1, EMB), _full_spec(1, EMB)],
    out_specs=[_row_spec(EMB), _full_spec(8, EMB)],
    out_shape=[jax.ShapeDtypeStruct((NN, EMB), jnp.float32),
               jax.ShapeDtypeStruct((8, EMB), jnp.float32)],
    scratch_shapes=[pltpu.VMEM((8, EMB), jnp.float32)],
)

_tc3 = pl.pallas_call(
    _tc3_body,
    grid=(GRID,),
    in_specs=[_row_spec(EMB), _full_spec(8, EMB), _row_spec(1), _row_spec(1),
              _full_spec(EMB, EMB)],
    out_specs=[_row_spec(HALF), _row_spec(HALF)],
    out_shape=[jax.ShapeDtypeStruct((NN, HALF), jnp.float32)] * 2,
)


def kernel(edge_index, x, src_nodes, dst_nodes, user_emb, book_emb,
           W1, b1, g1, be1, W2, b2, g2, be2):
    h = jnp.concatenate([user_emb, book_emb], axis=0)
    pad = EPAD - E
    srcp = jnp.concatenate([edge_index[0], jnp.zeros((pad,), jnp.int32)])
    dstp = jnp.concatenate(
        [edge_index[1], jnp.full((pad,), NN, jnp.int32)]).reshape(EPAD // 128, 128)
    d0, d1 = _deg_kernel(dstp)
    dp0 = d0[:NN].reshape(NN, 1)
    dp1 = d1[:NN].reshape(NN, 1)

    q1a, q1b = _tc1(h, dp0, dp1, W1)
    s1a, s1b = _mp_kernel(q1a, q1b, srcp, dstp)
    out1, ac1 = _tc2(s1a, s1b, dp0, dp1, b1.reshape(1, EMB),
                     g1.reshape(1, EMB), be1.reshape(1, EMB))
    q2a, q2b = _tc3(out1, ac1, dp0, dp1, W2)
    s2a, s2b = _mp_kernel(q2a, q2b, srcp, dstp)
    out2, ac2 = _tc2(s2a, s2b, dp0, dp1, b2.reshape(1, EMB),
                     g2.reshape(1, EMB), be2.reshape(1, EMB))
    ratings = _gd_kernel(out2, ac2, src_nodes.reshape(128, 128),
                         dst_nodes.reshape(128, 128))
    return ratings


# speedup vs baseline: 44.9552x; 1.5875x over previous
import functools

import jax
import jax.numpy as jnp
from jax import lax
from jax.experimental import pallas as pl
from jax.experimental.pallas import tpu as pltpu
from jax.experimental.pallas import tpu_sc as plsc

NN = 100000
NROWS = 100352
PK = NROWS // 8
PBLK = PK // 7
EMB = 32
HALF = 16
E = 1600000
NC, NS = 2, 16

CHUNK = 512
KJ = CHUNK // 128
EPT = 100352
NCH = EPT // CHUNK
EPAD = EPT * NS
ROWS_PT = NROWS // NS
ACC_ROWS = NROWS

DCHUNK = 1024
KJD = DCHUNK // 128
EPT_D = EPAD // (NC * NS)
NCH_D = EPT_D // DCHUNK
DEG_ROWS = NROWS
DROWS_PT = DEG_ROWS // NS

NP = 16384
PPT = NP // (NC * NS)
PJ = PPT // 128

_MESH = plsc.VectorSubcoreMesh(
    core_axis_name="c", subcore_axis_name="s", num_cores=NC, num_subcores=NS)
_SC_PARAMS = pltpu.CompilerParams(
    use_tc_tiling_on_sc=False, needs_layout_passes=False)


@functools.partial(
    pl.kernel,
    out_type=(jax.ShapeDtypeStruct((DEG_ROWS,), jnp.float32),
              jax.ShapeDtypeStruct((DEG_ROWS,), jnp.float32)),
    mesh=_MESH,
    compiler_params=_SC_PARAMS,
    scratch_types=[
        pltpu.VMEM_SHARED((DEG_ROWS,), jnp.float32),
        pltpu.VMEM((KJD, 128), jnp.int32),
        pltpu.VMEM((DROWS_PT,), jnp.float32),
        pltpu.VMEM((128,), jnp.float32),
        pltpu.SemaphoreType.DMA,
    ],
)
def _deg_kernel(dst_hbm, d0_hbm, d1_hbm, dacc, idx_d, zbuf, ones_v, sem):
    cid = lax.axis_index("c")
    sid = lax.axis_index("s")
    r0 = sid * DROWS_PT

    def zb(i, carry):
        zbuf[pl.ds(i * 16, 16)] = jnp.zeros((16,), jnp.float32)
        return carry

    lax.fori_loop(0, DROWS_PT // 16, zb, 0)

    def ob(i, carry):
        ones_v[pl.ds(i * 16, 16)] = jnp.ones((16,), jnp.float32)
        return carry

    lax.fori_loop(0, 8, ob, 0)
    pltpu.sync_copy(zbuf, dacc.at[pl.ds(r0, DROWS_PT)])
    plsc.subcore_barrier()
    wid = cid * NS + sid
    base128 = wid * (EPT_D // 128)

    def body(ch, carry):
        off = base128 + ch * KJD
        pltpu.async_copy(dst_hbm.at[pl.ds(off, KJD)], idx_d, sem).wait()
        for j in range(KJD):
            pltpu.sync_copy(ones_v, dacc.at[idx_d.at[j]], add=True)
        return carry

    lax.fori_loop(0, NCH_D, body, 0)
    plsc.subcore_barrier()
    pltpu.sync_copy(dacc.at[pl.ds(r0, DROWS_PT)], zbuf)

    @pl.when(cid == 0)
    def _():
        pltpu.sync_copy(zbuf, d0_hbm.at[pl.ds(r0, DROWS_PT)])

    @pl.when(cid == 1)
    def _():
        pltpu.sync_copy(zbuf, d1_hbm.at[pl.ds(r0, DROWS_PT)])


@functools.partial(
    pl.kernel,
    out_type=(jax.ShapeDtypeStruct((NROWS, HALF), jnp.float32),
              jax.ShapeDtypeStruct((NROWS, HALF), jnp.float32)),
    mesh=_MESH,
    compiler_params=_SC_PARAMS,
    scratch_types=[
        pltpu.VMEM_SHARED((ACC_ROWS, HALF), jnp.float32),
        pltpu.VMEM((2 * CHUNK,), jnp.int32),
        pltpu.VMEM((2 * KJ, 128), jnp.int32),
        pltpu.VMEM((CHUNK, HALF), jnp.float32),
        pltpu.VMEM((CHUNK, HALF), jnp.float32),
        pltpu.SemaphoreType.DMA,
        pltpu.SemaphoreType.DMA,
        pltpu.SemaphoreType.DMA,
        pltpu.SemaphoreType.DMA,
        pltpu.SemaphoreType.DMA,
    ],
)
def _mp_kernel(q0_hbm, q1_hbm, src_hbm, dst_hbm, a0_hbm, a1_hbm,
               acc, idx_s, idx_d, rows0, rows1,
               gsem0, gsem1, ssem0, ssem1, isem):
    cid = lax.axis_index("c")
    sid = lax.axis_index("s")
    r0 = sid * ROWS_PT
    chunks = (512,) * 12 + (ROWS_PT - 12 * 512,)

    def stage(hbm_ref, to_spmem):
        off = 0
        for cnt in chunks:
            hslice = hbm_ref.at[pl.ds(r0 + off, cnt)]
            vslice = rows0.at[pl.ds(0, cnt)]
            aslice = acc.at[pl.ds(r0 + off, cnt)]
            if to_spmem:
                pltpu.sync_copy(hslice, vslice)
                pltpu.sync_copy(vslice, aslice)
            else:
                pltpu.sync_copy(aslice, vslice)
                pltpu.sync_copy(vslice, hslice)
            off += cnt

    @pl.when(cid == 0)
    def _():
        stage(q0_hbm, True)

    @pl.when(cid == 1)
    def _():
        stage(q1_hbm, True)

    plsc.subcore_barrier()
    base128 = sid * (EPT // 128)

    def run(q_hbm):
        def fire_gather(rows_buf, sem, slot):
            pltpu.async_copy(q_hbm.at[idx_s.at[pl.ds(slot * CHUNK, CHUNK)]],
                             rows_buf, sem)

        def drain(sem):
            pltpu.make_async_copy(q_hbm.at[pl.ds(0, CHUNK)], rows0,
                                  sem).wait()

        def fire_scatter(rows_buf, sem, j0):
            for j in range(KJ):
                pltpu.async_copy(rows_buf.at[pl.ds(j * 128, 128)],
                                 acc.at[idx_d.at[j0 + j]], sem, add=True)

        def body(chp, carry):
            off = base128 + chp * (2 * KJ)
            c1 = pltpu.async_copy(
                src_hbm.at[pl.ds(off * 128, 2 * CHUNK)], idx_s, isem)
            c2 = pltpu.async_copy(dst_hbm.at[pl.ds(off, 2 * KJ)], idx_d, isem)
            c1.wait()
            c2.wait()
            fire_gather(rows0, gsem0, 0)
            fire_gather(rows1, gsem1, 1)
            drain(gsem0)
            fire_scatter(rows0, ssem0, 0)
            drain(gsem1)
            fire_scatter(rows1, ssem1, KJ)
            drain(ssem0)
            drain(ssem1)
            return carry

        lax.fori_loop(0, NCH // 2, body, 0)

    @pl.when(cid == 0)
    def _():
        run(q0_hbm)

    @pl.when(cid == 1)
    def _():
        run(q1_hbm)

    plsc.subcore_barrier()

    @pl.when(cid == 0)
    def _():
        stage(a0_hbm, False)

    @pl.when(cid == 1)
    def _():
        stage(a1_hbm, False)


@functools.partial(
    pl.kernel,
    out_type=jax.ShapeDtypeStruct((NP,), jnp.float32),
    mesh=_MESH,
    compiler_params=_SC_PARAMS,
    scratch_types=[
        pltpu.VMEM((PJ, 128), jnp.int32),
        pltpu.VMEM((PJ, 128), jnp.int32),
        pltpu.VMEM((128, HALF), jnp.float32),
        pltpu.VMEM((128, HALF), jnp.float32),
        pltpu.VMEM((128, HALF), jnp.float32),
        pltpu.VMEM((128, HALF), jnp.float32),
        pltpu.VMEM((128, HALF), jnp.float32),
        pltpu.VMEM((128, HALF), jnp.float32),
        pltpu.VMEM((8, EMB), jnp.float32),
        pltpu.VMEM((PPT,), jnp.float32),
        pltpu.SemaphoreType.DMA,
    ],
)
def _gd_kernel(sa_hbm, sb_hbm, dv_hbm, ac_hbm, srcn_hbm, dstn_hbm, r_hbm,
               idx_a, idx_b, bufsa, bufsb, bufda, bufdb, dvs, dvd,
               ac_v, res, sem):
    cid = lax.axis_index("c")
    sid = lax.axis_index("s")
    wid = cid * NS + sid
    pltpu.sync_copy(ac_hbm, ac_v)
    pltpu.sync_copy(srcn_hbm.at[pl.ds(wid * PJ, PJ)], idx_a)
    pltpu.sync_copy(dstn_hbm.at[pl.ds(wid * PJ, PJ)], idx_b)
    lane = lax.iota(jnp.int32, 16)
    zero16 = jnp.zeros((16,), jnp.int32)
    one16 = zero16 + 1

    def jbody(j, carry):
        cps = [
            pltpu.async_copy(sa_hbm.at[idx_a.at[j]], bufsa, sem),
            pltpu.async_copy(sb_hbm.at[idx_a.at[j]], bufsb, sem),
            pltpu.async_copy(dv_hbm.at[idx_a.at[j]], dvs, sem),
            pltpu.async_copy(sa_hbm.at[idx_b.at[j]], bufda, sem),
            pltpu.async_copy(sb_hbm.at[idx_b.at[j]], bufdb, sem),
            pltpu.async_copy(dv_hbm.at[idx_b.at[j]], dvd, sem),
        ]
        for cp in cps:
            cp.wait()

        def gbody(g, carry2):
            rowi = g * 16 + lane
            acc16 = jnp.zeros((16,), jnp.float32)
            for h, (bs, bd) in enumerate(((bufsa, bufda), (bufsb, bufdb))):
                def cbody(cc, acc_in, bs=bs, bd=bd, h=h):
                    col = zero16 + cc
                    a_c = plsc.load_gather(ac_v, [zero16, col + h * HALF])
                    c_c = plsc.load_gather(ac_v, [one16, col + h * HALF])
                    va = plsc.load_gather(bs, [rowi, col])
                    vb = plsc.load_gather(bd, [rowi, col])
                    da = plsc.load_gather(dvs, [rowi, col])
                    db = plsc.load_gather(dvd, [rowi, col])
                    ya = jnp.maximum(a_c * (va * da) + c_c, 0.0)
                    yb = jnp.maximum(a_c * (vb * db) + c_c, 0.0)
                    return acc_in + ya * yb

                acc16 = lax.fori_loop(0, HALF, cbody, acc16)
            res[pl.ds(j * 128 + g * 16, 16)] = acc16
            return carry2

        lax.fori_loop(0, 8, gbody, 0)
        return carry

    lax.fori_loop(0, PJ, jbody, 0)
    pltpu.sync_copy(res, r_hbm.at[pl.ds(wid * PPT, PPT)])


NN_F = float(NN)


def _tca_body(ha_ref, hb_ref, dp0_ref, dp1_ref, r_ref,
              waa_ref, wab_ref, wba_ref, wbb_ref,
              qa_ref, qb_ref, dv_ref):
    deg = dp0_ref[...] + dp1_ref[...] + 1.0
    dinvp = jnp.dot(lax.rsqrt(deg), r_ref[...],
                    preferred_element_type=jnp.float32,
                    precision=lax.Precision.HIGHEST)
    dv_ref[...] = dinvp
    ha = ha_ref[...]
    hb = hb_ref[...]
    dot = functools.partial(jnp.dot, preferred_element_type=jnp.float32)
    qa_ref[...] = (dot(ha, waa_ref[...]) + dot(hb, wba_ref[...])) * dinvp
    qb_ref[...] = (dot(ha, wab_ref[...]) + dot(hb, wbb_ref[...])) * dinvp


def _tcb_body(sa_ref, sb_ref, dp0_ref, dp1_ref, r_ref, f_ref, ft_ref,
              bn_ref, waa_ref, wab_ref, wba_ref, wbb_ref,
              qa_ref, qb_ref, accum):
    p = pl.program_id(0)
    i = pl.program_id(1)
    dot = functools.partial(jnp.dot, preferred_element_type=jnp.float32)
    dotx = functools.partial(jnp.dot, preferred_element_type=jnp.float32,
                             precision=lax.Precision.HIGHEST)
    deg = dp0_ref[...] + dp1_ref[...] + 1.0
    dinvp = dotx(lax.rsqrt(deg), r_ref[...])
    bap = dotx(bn_ref[0:1, :], ft_ref[...])
    bbp = dotx(bn_ref[1:2, :], ft_ref[...])
    outa = sa_ref[...] * dinvp + bap
    outb = sb_ref[...] * dinvp + bbp

    @pl.when(p == 0)
    def _():
        @pl.when(i == 0)
        def _():
            accum[...] = jnp.zeros_like(accum)

        rowid = lax.broadcasted_iota(jnp.int32, (PBLK, 128), 0) + i * PBLK
        m = rowid < (NN // 8)
        oa = jnp.where(m, outa, 0.0)
        ob = jnp.where(m, outb, 0.0)
        accum[0:1, :] += jnp.sum(oa, axis=0, keepdims=True)
        accum[1:2, :] += jnp.sum(oa * oa, axis=0, keepdims=True)
        accum[2:3, :] += jnp.sum(ob, axis=0, keepdims=True)
        accum[3:4, :] += jnp.sum(ob * ob, axis=0, keepdims=True)

        @pl.when(i == 6)
        def _():
            mu_a = dotx(accum[0:1, :], f_ref[...]) / NN_F
            var_a = dotx(accum[1:2, :], f_ref[...]) / NN_F - mu_a * mu_a
            mu_b = dotx(accum[2:3, :], f_ref[...]) / NN_F
            var_b = dotx(accum[3:4, :], f_ref[...]) / NN_F - mu_b * mu_b
            a16a = bn_ref[2:3, :] * lax.rsqrt(var_a + 1e-5)
            c16a = bn_ref[4:5, :] - a16a * mu_a
            a16b = bn_ref[3:4, :] * lax.rsqrt(var_b + 1e-5)
            c16b = bn_ref[5:6, :] - a16b * mu_b
            accum[4:5, :] = dotx(a16a, ft_ref[...])
            accum[5:6, :] = dotx(c16a, ft_ref[...])
            accum[6:7, :] = dotx(a16b, ft_ref[...])
            accum[7:8, :] = dotx(c16b, ft_ref[...])

    @pl.when(p == 1)
    def _():
        h1a = jnp.maximum(accum[4:5, :] * outa + accum[5:6, :], 0.0)
        h1b = jnp.maximum(accum[6:7, :] * outb + accum[7:8, :], 0.0)
        qa_ref[...] = (dot(h1a, waa_ref[...]) + dot(h1b, wba_ref[...])) * dinvp
        qb_ref[...] = (dot(h1a, wab_ref[...]) + dot(h1b, wbb_ref[...])) * dinvp


def _tcc_body(sa_ref, sb_ref, dp0_ref, dp1_ref, r_ref, f_ref, ft_ref,
              bn_ref, ac_ref, accum):
    i = pl.program_id(0)
    dot = functools.partial(jnp.dot, preferred_element_type=jnp.float32)
    dotx = functools.partial(jnp.dot, preferred_element_type=jnp.float32,
                             precision=lax.Precision.HIGHEST)
    deg = dp0_ref[...] + dp1_ref[...] + 1.0
    dinvp = dotx(lax.rsqrt(deg), r_ref[...])
    bap = dotx(bn_ref[0:1, :], ft_ref[...])
    bbp = dotx(bn_ref[1:2, :], ft_ref[...])
    outa = sa_ref[...] * dinvp + bap
    outb = sb_ref[...] * dinvp + bbp

    @pl.when(i == 0)
    def _():
        accum[...] = jnp.zeros_like(accum)

    rowid = lax.broadcasted_iota(jnp.int32, (PBLK, 128), 0) + i * PBLK
    m = rowid < (NN // 8)
    oa = jnp.where(m, outa, 0.0)
    ob = jnp.where(m, outb, 0.0)
    accum[0:1, :] += jnp.sum(oa, axis=0, keepdims=True)
    accum[1:2, :] += jnp.sum(oa * oa, axis=0, keepdims=True)
    accum[2:3, :] += jnp.sum(ob, axis=0, keepdims=True)
    accum[3:4, :] += jnp.sum(ob * ob, axis=0, keepdims=True)

    @pl.when(i == 6)
    def _():
        mu_a = dotx(accum[0:1, :], f_ref[...]) / NN_F
        var_a = dotx(accum[1:2, :], f_ref[...]) / NN_F - mu_a * mu_a
        mu_b = dotx(accum[2:3, :], f_ref[...]) / NN_F
        var_b = dotx(accum[3:4, :], f_ref[...]) / NN_F - mu_b * mu_b
        a16a = bn_ref[2:3, :] * lax.rsqrt(var_a + 1e-5)
        c16a = bn_ref[4:5, :] - a16a * mu_a
        a16b = bn_ref[3:4, :] * lax.rsqrt(var_b + 1e-5)
        c16b = bn_ref[5:6, :] - a16b * mu_b
        ca = c16a + a16a * bn_ref[0:1, :]
        cb = c16b + a16b * bn_ref[1:2, :]
        ac_ref[0:1, :] = jnp.concatenate([a16a, a16b], axis=1)
        ac_ref[1:2, :] = jnp.concatenate([ca, cb], axis=1)
        ac_ref[2:8, :] = jnp.zeros((6, EMB), jnp.float32)


def _pblk_spec():
    return pl.BlockSpec((PBLK, 128), lambda i: (i, 0))


def _pblk8_spec():
    return pl.BlockSpec((PBLK, 8), lambda i: (i, 0))


def _cst(rows, cols):
    return pl.BlockSpec((rows, cols), lambda i: (0, 0))


_tca = pl.pallas_call(
    _tca_body,
    grid=(7,),
    in_specs=[_pblk_spec(), _pblk_spec(), _pblk8_spec(), _pblk8_spec(),
              _cst(8, 128), _cst(128, 128), _cst(128, 128), _cst(128, 128),
              _cst(128, 128)],
    out_specs=[_pblk_spec(), _pblk_spec(), _pblk_spec()],
    out_shape=[jax.ShapeDtypeStruct((PK, 128), jnp.float32)] * 3,
)


def _pblk2_spec():
    return pl.BlockSpec((PBLK, 128), lambda p, i: (i, 0))


def _pblk28_spec():
    return pl.BlockSpec((PBLK, 8), lambda p, i: (i, 0))


def _cst2(rows, cols):
    return pl.BlockSpec((rows, cols), lambda p, i: (0, 0))


_tcb = pl.pallas_call(
    _tcb_body,
    grid=(2, 7),
    in_specs=[_pblk2_spec(), _pblk2_spec(), _pblk28_spec(), _pblk28_spec(),
              _cst2(8, 128), _cst2(128, HALF), _cst2(HALF, 128), _cst2(8, HALF),
              _cst2(128, 128), _cst2(128, 128), _cst2(128, 128),
              _cst2(128, 128)],
    out_specs=[_pblk2_spec(), _pblk2_spec()],
    out_shape=[jax.ShapeDtypeStruct((PK, 128), jnp.float32)] * 2,
    scratch_shapes=[pltpu.VMEM((8, 128), jnp.float32)],
)

_tcc = pl.pallas_call(
    _tcc_body,
    grid=(7,),
    in_specs=[_pblk_spec(), _pblk_spec(), _pblk8_spec(), _pblk8_spec(),
              _cst(8, 128), _cst(128, HALF), _cst(HALF, 128), _cst(8, HALF)],
    out_specs=_cst(8, EMB),
    out_shape=jax.ShapeDtypeStruct((8, EMB), jnp.float32),
    scratch_shapes=[pltpu.VMEM((8, 128), jnp.float32)],
)


def kernel(edge_index, x, src_nodes, dst_nodes, user_emb, book_emb,
           W1, b1, g1, be1, W2, b2, g2, be2):
    f32 = jnp.float32
    zrows = jnp.zeros((NROWS - NN, HALF), f32)
    ha_p = jnp.concatenate(
        [user_emb[:, :HALF], book_emb[:, :HALF], zrows], 0).reshape(PK, 128)
    hb_p = jnp.concatenate(
        [user_emb[:, HALF:], book_emb[:, HALF:], zrows], 0).reshape(PK, 128)
    pad = EPAD - E
    srcp = jnp.concatenate([edge_index[0], jnp.zeros((pad,), jnp.int32)])
    dstp = jnp.concatenate(
        [edge_index[1], jnp.full((pad,), NN, jnp.int32)]).reshape(
            EPAD // 128, 128)

    i128 = jnp.arange(128)
    rmat = (i128[None, :] // HALF == jnp.arange(8)[:, None]).astype(f32)
    fmat = ((i128 % HALF)[:, None] == jnp.arange(HALF)[None, :]).astype(f32)
    ftmat = fmat.T

    eye8 = jnp.eye(8, dtype=f32)

    def bd(w, si, so):
        return jnp.kron(
            eye8, w[HALF * si:HALF * (si + 1), HALF * so:HALF * (so + 1)])

    def bnmat(b, g, be):
        return jnp.concatenate(
            [b.reshape(2, HALF), g.reshape(2, HALF), be.reshape(2, HALF),
             jnp.zeros((2, HALF), f32)], 0)

    d0, d1 = _deg_kernel(dstp)
    dp0 = d0.reshape(PK, 8)
    dp1 = d1.reshape(PK, 8)

    q1a, q1b, dv = _tca(ha_p, hb_p, dp0, dp1, rmat,
                        bd(W1, 0, 0), bd(W1, 0, 1), bd(W1, 1, 0), bd(W1, 1, 1))
    s1a, s1b = _mp_kernel(q1a.reshape(NROWS, HALF), q1b.reshape(NROWS, HALF),
                          srcp, dstp)
    q2a, q2b = _tcb(s1a.reshape(PK, 128), s1b.reshape(PK, 128), dp0, dp1,
                    rmat, fmat, ftmat, bnmat(b1, g1, be1),
                    bd(W2, 0, 0), bd(W2, 0, 1), bd(W2, 1, 0), bd(W2, 1, 1))
    s2a, s2b = _mp_kernel(q2a.reshape(NROWS, HALF), q2b.reshape(NROWS, HALF),
                          srcp, dstp)
    ac2 = _tcc(s2a.reshape(PK, 128), s2b.reshape(PK, 128), dp0, dp1,
               rmat, fmat, ftmat, bnmat(b2, g2, be2))
    ratings = _gd_kernel(s2a, s2b, dv.reshape(NROWS, HALF), ac2,
                         src_nodes.reshape(128, 128),
                         dst_nodes.reshape(128, 128))
    return ratings


# speedup vs baseline: 45.1486x; 1.0043x over previous
import functools

import jax
import jax.numpy as jnp
from jax import lax
from jax.experimental import pallas as pl
from jax.experimental.pallas import tpu as pltpu
from jax.experimental.pallas import tpu_sc as plsc

NN = 100000
NROWS = 100352
PK = NROWS // 8
PBLK = PK // 7
EMB = 32
HALF = 16
E = 1600000
NC, NS = 2, 16

CHUNK = 256
KJ = CHUNK // 128
EPT = 100352
NCH = EPT // (4 * CHUNK)
EPAD = EPT * NS
ROWS_PT = NROWS // NS
ACC_ROWS = NROWS

DCHUNK = 1024
KJD = DCHUNK // 128
EPT_D = EPAD // (NC * NS)
NCH_D = EPT_D // DCHUNK
DEG_ROWS = NROWS
DROWS_PT = DEG_ROWS // NS

NP = 16384
PPT = NP // (NC * NS)
PJ = PPT // 128

_MESH = plsc.VectorSubcoreMesh(
    core_axis_name="c", subcore_axis_name="s", num_cores=NC, num_subcores=NS)
_SC_PARAMS = pltpu.CompilerParams(
    use_tc_tiling_on_sc=False, needs_layout_passes=False)


@functools.partial(
    pl.kernel,
    out_type=(jax.ShapeDtypeStruct((DEG_ROWS,), jnp.float32),
              jax.ShapeDtypeStruct((DEG_ROWS,), jnp.float32)),
    mesh=_MESH,
    compiler_params=_SC_PARAMS,
    scratch_types=[
        pltpu.VMEM_SHARED((DEG_ROWS,), jnp.float32),
        pltpu.VMEM((KJD, 128), jnp.int32),
        pltpu.VMEM((DROWS_PT,), jnp.float32),
        pltpu.VMEM((128,), jnp.float32),
        pltpu.SemaphoreType.DMA,
        pltpu.SemaphoreType.DMA,
    ],
)
def _deg_kernel(dst_hbm, d0_hbm, d1_hbm, dacc, idx_d, zbuf, ones_v, sem,
                ssem):
    cid = lax.axis_index("c")
    sid = lax.axis_index("s")
    r0 = sid * DROWS_PT

    def zb(i, carry):
        zbuf[pl.ds(i * 16, 16)] = jnp.zeros((16,), jnp.float32)
        return carry

    lax.fori_loop(0, DROWS_PT // 16, zb, 0)

    def ob(i, carry):
        ones_v[pl.ds(i * 16, 16)] = jnp.ones((16,), jnp.float32)
        return carry

    lax.fori_loop(0, 8, ob, 0)
    pltpu.sync_copy(zbuf, dacc.at[pl.ds(r0, DROWS_PT)])
    plsc.subcore_barrier()
    wid = cid * NS + sid
    base128 = wid * (EPT_D // 128)

    def body(ch, carry):
        off = base128 + ch * KJD
        pltpu.async_copy(dst_hbm.at[pl.ds(off, KJD)], idx_d, sem).wait()
        for j in range(KJD):
            pltpu.async_copy(ones_v, dacc.at[idx_d.at[j]], ssem, add=True)
        pltpu.make_async_copy(d0_hbm.at[pl.ds(0, DCHUNK)],
                              zbuf.at[pl.ds(0, DCHUNK)], ssem).wait()
        return carry

    lax.fori_loop(0, NCH_D, body, 0)
    plsc.subcore_barrier()
    pltpu.sync_copy(dacc.at[pl.ds(r0, DROWS_PT)], zbuf)

    @pl.when(cid == 0)
    def _():
        pltpu.sync_copy(zbuf, d0_hbm.at[pl.ds(r0, DROWS_PT)])

    @pl.when(cid == 1)
    def _():
        pltpu.sync_copy(zbuf, d1_hbm.at[pl.ds(r0, DROWS_PT)])


@functools.partial(
    pl.kernel,
    out_type=(jax.ShapeDtypeStruct((NROWS, HALF), jnp.float32),
              jax.ShapeDtypeStruct((NROWS, HALF), jnp.float32)),
    mesh=_MESH,
    compiler_params=_SC_PARAMS,
    scratch_types=[
        pltpu.VMEM_SHARED((ACC_ROWS, HALF), jnp.float32),
        pltpu.VMEM((2 * CHUNK,), jnp.int32),
        pltpu.VMEM((2 * CHUNK,), jnp.int32),
        pltpu.VMEM((4, 128), jnp.int32),
        pltpu.VMEM((4, 128), jnp.int32),
        pltpu.VMEM((CHUNK, HALF), jnp.float32),
        pltpu.VMEM((CHUNK, HALF), jnp.float32),
        pltpu.VMEM((CHUNK, HALF), jnp.float32),
        pltpu.VMEM((CHUNK, HALF), jnp.float32),
        pltpu.SemaphoreType.DMA,
        pltpu.SemaphoreType.DMA,
        pltpu.SemaphoreType.DMA,
        pltpu.SemaphoreType.DMA,
        pltpu.SemaphoreType.DMA,
        pltpu.SemaphoreType.DMA,
        pltpu.SemaphoreType.DMA,
    ],
)
def _mp_kernel(q0_hbm, q1_hbm, src_hbm, dst_hbm, a0_hbm, a1_hbm,
               acc, ixsx, ixsy, ixdx, ixdy, ra, rb, rc, rd,
               gsa, gsb, gsc, gsd, ssx, ssy, isem):
    cid = lax.axis_index("c")
    sid = lax.axis_index("s")
    r0 = sid * ROWS_PT
    chunks = (CHUNK,) * 24 + (ROWS_PT - 24 * CHUNK,)

    def stage(hbm_ref, to_spmem):
        off = 0
        for cnt in chunks:
            hslice = hbm_ref.at[pl.ds(r0 + off, cnt)]
            vslice = ra.at[pl.ds(0, cnt)]
            aslice = acc.at[pl.ds(r0 + off, cnt)]
            if to_spmem:
                pltpu.sync_copy(hslice, vslice)
                pltpu.sync_copy(vslice, aslice)
            else:
                pltpu.sync_copy(aslice, vslice)
                pltpu.sync_copy(vslice, hslice)
            off += cnt

    @pl.when(cid == 0)
    def _():
        stage(q0_hbm, True)

    @pl.when(cid == 1)
    def _():
        stage(q1_hbm, True)

    plsc.subcore_barrier()
    base128 = sid * (EPT // 128)

    def run(q_hbm):
        def fire_gather(rows_buf, sem, ixs, slot):
            pltpu.async_copy(q_hbm.at[ixs.at[pl.ds(slot * CHUNK, CHUNK)]],
                             rows_buf, sem)

        def drain(sem):
            pltpu.make_async_copy(q_hbm.at[pl.ds(0, CHUNK)], ra, sem).wait()

        def fire_scatter(rows_buf, sem, ixd, j0):
            for j in range(KJ):
                pltpu.async_copy(rows_buf.at[pl.ds(j * 128, 128)],
                                 acc.at[ixd.at[j0 + j]], sem, add=True)

        def copy_idx(off2d, ixs, ixd):
            c1 = pltpu.async_copy(
                src_hbm.at[pl.ds(off2d * 128, 2 * CHUNK)], ixs, isem)
            c2 = pltpu.async_copy(dst_hbm.at[pl.ds(off2d, 2 * KJ)], ixd, isem)
            c1.wait()
            c2.wait()

        def body(i, carry):
            off = base128 + i * (4 * KJ)
            copy_idx(off, ixsx, ixdx)
            fire_gather(ra, gsa, ixsx, 0)
            fire_gather(rb, gsb, ixsx, 1)

            @pl.when(i > 0)
            def _():
                drain(ssy)
                drain(ssy)

            copy_idx(off + 2 * KJ, ixsy, ixdy)
            drain(gsa)
            fire_scatter(ra, ssx, ixdx, 0)
            drain(gsb)
            fire_scatter(rb, ssx, ixdx, KJ)
            fire_gather(rc, gsc, ixsy, 0)
            fire_gather(rd, gsd, ixsy, 1)
            drain(ssx)
            drain(ssx)
            drain(gsc)
            fire_scatter(rc, ssy, ixdy, 0)
            drain(gsd)
            fire_scatter(rd, ssy, ixdy, KJ)
            return carry

        lax.fori_loop(0, NCH, body, 0)
        drain(ssy)
        drain(ssy)

    @pl.when(cid == 0)
    def _():
        run(q0_hbm)

    @pl.when(cid == 1)
    def _():
        run(q1_hbm)

    plsc.subcore_barrier()

    @pl.when(cid == 0)
    def _():
        stage(a0_hbm, False)

    @pl.when(cid == 1)
    def _():
        stage(a1_hbm, False)


@functools.partial(
    pl.kernel,
    out_type=jax.ShapeDtypeStruct((NP,), jnp.float32),
    mesh=_MESH,
    compiler_params=_SC_PARAMS,
    scratch_types=[
        pltpu.VMEM((PJ, 128), jnp.int32),
        pltpu.VMEM((PJ, 128), jnp.int32),
        pltpu.VMEM((128, HALF), jnp.float32),
        pltpu.VMEM((128, HALF), jnp.float32),
        pltpu.VMEM((128, HALF), jnp.float32),
        pltpu.VMEM((128, HALF), jnp.float32),
        pltpu.VMEM((128, HALF), jnp.float32),
        pltpu.VMEM((128, HALF), jnp.float32),
        pltpu.VMEM((8, EMB), jnp.float32),
        pltpu.VMEM((PPT,), jnp.float32),
        pltpu.SemaphoreType.DMA,
    ],
)
def _gd_kernel(sa_hbm, sb_hbm, dv_hbm, ac_hbm, srcn_hbm, dstn_hbm, r_hbm,
               idx_a, idx_b, bufsa, bufsb, bufda, bufdb, dvs, dvd,
               ac_v, res, sem):
    cid = lax.axis_index("c")
    sid = lax.axis_index("s")
    wid = cid * NS + sid
    pltpu.sync_copy(ac_hbm, ac_v)
    pltpu.sync_copy(srcn_hbm.at[pl.ds(wid * PJ, PJ)], idx_a)
    pltpu.sync_copy(dstn_hbm.at[pl.ds(wid * PJ, PJ)], idx_b)
    lane = lax.iota(jnp.int32, 16)
    zero16 = jnp.zeros((16,), jnp.int32)
    one16 = zero16 + 1

    def jbody(j, carry):
        cps = [
            pltpu.async_copy(sa_hbm.at[idx_a.at[j]], bufsa, sem),
            pltpu.async_copy(sb_hbm.at[idx_a.at[j]], bufsb, sem),
            pltpu.async_copy(dv_hbm.at[idx_a.at[j]], dvs, sem),
            pltpu.async_copy(sa_hbm.at[idx_b.at[j]], bufda, sem),
            pltpu.async_copy(sb_hbm.at[idx_b.at[j]], bufdb, sem),
            pltpu.async_copy(dv_hbm.at[idx_b.at[j]], dvd, sem),
        ]
        for cp in cps:
            cp.wait()

        def gbody(g, carry2):
            rowi = g * 16 + lane
            acc16 = jnp.zeros((16,), jnp.float32)
            for h, (bs, bd) in enumerate(((bufsa, bufda), (bufsb, bufdb))):
                def cbody(cc, acc_in, bs=bs, bd=bd, h=h):
                    col = zero16 + cc
                    a_c = plsc.load_gather(ac_v, [zero16, col + h * HALF])
                    c_c = plsc.load_gather(ac_v, [one16, col + h * HALF])
                    va = plsc.load_gather(bs, [rowi, col])
                    vb = plsc.load_gather(bd, [rowi, col])
                    da = plsc.load_gather(dvs, [rowi, col])
                    db = plsc.load_gather(dvd, [rowi, col])
                    ya = jnp.maximum(a_c * (va * da) + c_c, 0.0)
                    yb = jnp.maximum(a_c * (vb * db) + c_c, 0.0)
                    return acc_in + ya * yb

                acc16 = lax.fori_loop(0, HALF, cbody, acc16)
            res[pl.ds(j * 128 + g * 16, 16)] = acc16
            return carry2

        lax.fori_loop(0, 8, gbody, 0)
        return carry

    lax.fori_loop(0, PJ, jbody, 0)
    pltpu.sync_copy(res, r_hbm.at[pl.ds(wid * PPT, PPT)])


NN_F = float(NN)


def _tca_body(ha_ref, hb_ref, dp0_ref, dp1_ref, r_ref,
              waa_ref, wab_ref, wba_ref, wbb_ref,
              qa_ref, qb_ref, dv_ref):
    deg = dp0_ref[...] + dp1_ref[...] + 1.0
    dinvp = jnp.dot(lax.rsqrt(deg), r_ref[...],
                    preferred_element_type=jnp.float32,
                    precision=lax.Precision.HIGHEST)
    dv_ref[...] = dinvp
    ha = ha_ref[...]
    hb = hb_ref[...]
    dot = functools.partial(jnp.dot, preferred_element_type=jnp.float32)
    qa_ref[...] = (dot(ha, waa_ref[...]) + dot(hb, wba_ref[...])) * dinvp
    qb_ref[...] = (dot(ha, wab_ref[...]) + dot(hb, wbb_ref[...])) * dinvp


def _tcb_body(sa_ref, sb_ref, dp0_ref, dp1_ref, r_ref, f_ref, ft_ref,
              bn_ref, waa_ref, wab_ref, wba_ref, wbb_ref,
              qa_ref, qb_ref, accum):
    p = pl.program_id(0)
    i = pl.program_id(1)
    dot = functools.partial(jnp.dot, preferred_element_type=jnp.float32)
    dotx = functools.partial(jnp.dot, preferred_element_type=jnp.float32,
                             precision=lax.Precision.HIGHEST)
    deg = dp0_ref[...] + dp1_ref[...] + 1.0
    dinvp = dotx(lax.rsqrt(deg), r_ref[...])
    bap = dotx(bn_ref[0:1, :], ft_ref[...])
    bbp = dotx(bn_ref[1:2, :], ft_ref[...])
    outa = sa_ref[...] * dinvp + bap
    outb = sb_ref[...] * dinvp + bbp

    @pl.when(p == 0)
    def _():
        @pl.when(i == 0)
        def _():
            accum[...] = jnp.zeros_like(accum)

        rowid = lax.broadcasted_iota(jnp.int32, (PBLK, 128), 0) + i * PBLK
        m = rowid < (NN // 8)
        oa = jnp.where(m, outa, 0.0)
        ob = jnp.where(m, outb, 0.0)
        accum[0:1, :] += jnp.sum(oa, axis=0, keepdims=True)
        accum[1:2, :] += jnp.sum(oa * oa, axis=0, keepdims=True)
        accum[2:3, :] += jnp.sum(ob, axis=0, keepdims=True)
        accum[3:4, :] += jnp.sum(ob * ob, axis=0, keepdims=True)

        @pl.when(i == 6)
        def _():
            mu_a = dotx(accum[0:1, :], f_ref[...]) / NN_F
            var_a = dotx(accum[1:2, :], f_ref[...]) / NN_F - mu_a * mu_a
            mu_b = dotx(accum[2:3, :], f_ref[...]) / NN_F
            var_b = dotx(accum[3:4, :], f_ref[...]) / NN_F - mu_b * mu_b
            a16a = bn_ref[2:3, :] * lax.rsqrt(var_a + 1e-5)
            c16a = bn_ref[4:5, :] - a16a * mu_a
            a16b = bn_ref[3:4, :] * lax.rsqrt(var_b + 1e-5)
            c16b = bn_ref[5:6, :] - a16b * mu_b
            accum[4:5, :] = dotx(a16a, ft_ref[...])
            accum[5:6, :] = dotx(c16a, ft_ref[...])
            accum[6:7, :] = dotx(a16b, ft_ref[...])
            accum[7:8, :] = dotx(c16b, ft_ref[...])

    @pl.when(p == 1)
    def _():
        h1a = jnp.maximum(accum[4:5, :] * outa + accum[5:6, :], 0.0)
        h1b = jnp.maximum(accum[6:7, :] * outb + accum[7:8, :], 0.0)
        qa_ref[...] = (dot(h1a, waa_ref[...]) + dot(h1b, wba_ref[...])) * dinvp
        qb_ref[...] = (dot(h1a, wab_ref[...]) + dot(h1b, wbb_ref[...])) * dinvp


def _tcc_body(sa_ref, sb_ref, dp0_ref, dp1_ref, r_ref, f_ref, ft_ref,
              bn_ref, ac_ref, accum):
    i = pl.program_id(0)
    dot = functools.partial(jnp.dot, preferred_element_type=jnp.float32)
    dotx = functools.partial(jnp.dot, preferred_element_type=jnp.float32,
                             precision=lax.Precision.HIGHEST)
    deg = dp0_ref[...] + dp1_ref[...] + 1.0
    dinvp = dotx(lax.rsqrt(deg), r_ref[...])
    bap = dotx(bn_ref[0:1, :], ft_ref[...])
    bbp = dotx(bn_ref[1:2, :], ft_ref[...])
    outa = sa_ref[...] * dinvp + bap
    outb = sb_ref[...] * dinvp + bbp

    @pl.when(i == 0)
    def _():
        accum[...] = jnp.zeros_like(accum)

    rowid = lax.broadcasted_iota(jnp.int32, (PBLK, 128), 0) + i * PBLK
    m = rowid < (NN // 8)
    oa = jnp.where(m, outa, 0.0)
    ob = jnp.where(m, outb, 0.0)
    accum[0:1, :] += jnp.sum(oa, axis=0, keepdims=True)
    accum[1:2, :] += jnp.sum(oa * oa, axis=0, keepdims=True)
    accum[2:3, :] += jnp.sum(ob, axis=0, keepdims=True)
    accum[3:4, :] += jnp.sum(ob * ob, axis=0, keepdims=True)

    @pl.when(i == 6)
    def _():
        mu_a = dotx(accum[0:1, :], f_ref[...]) / NN_F
        var_a = dotx(accum[1:2, :], f_ref[...]) / NN_F - mu_a * mu_a
        mu_b = dotx(accum[2:3, :], f_ref[...]) / NN_F
        var_b = dotx(accum[3:4, :], f_ref[...]) / NN_F - mu_b * mu_b
        a16a = bn_ref[2:3, :] * lax.rsqrt(var_a + 1e-5)
        c16a = bn_ref[4:5, :] - a16a * mu_a
        a16b = bn_ref[3:4, :] * lax.rsqrt(var_b + 1e-5)
        c16b = bn_ref[5:6, :] - a16b * mu_b
        ca = c16a + a16a * bn_ref[0:1, :]
        cb = c16b + a16b * bn_ref[1:2, :]
        ac_ref[0:1, :] = jnp.concatenate([a16a, a16b], axis=1)
        ac_ref[1:2, :] = jnp.concatenate([ca, cb], axis=1)
        ac_ref[2:8, :] = jnp.zeros((6, EMB), jnp.float32)


def _pblk_spec():
    return pl.BlockSpec((PBLK, 128), lambda i: (i, 0))


def _pblk8_spec():
    return pl.BlockSpec((PBLK, 8), lambda i: (i, 0))


def _cst(rows, cols):
    return pl.BlockSpec((rows, cols), lambda i: (0, 0))


_tca = pl.pallas_call(
    _tca_body,
    grid=(7,),
    in_specs=[_pblk_spec(), _pblk_spec(), _pblk8_spec(), _pblk8_spec(),
              _cst(8, 128), _cst(128, 128), _cst(128, 128), _cst(128, 128),
              _cst(128, 128)],
    out_specs=[_pblk_spec(), _pblk_spec(), _pblk_spec()],
    out_shape=[jax.ShapeDtypeStruct((PK, 128), jnp.float32)] * 3,
)


def _pblk2_spec():
    return pl.BlockSpec((PBLK, 128), lambda p, i: (i, 0))


def _pblk28_spec():
    return pl.BlockSpec((PBLK, 8), lambda p, i: (i, 0))


def _cst2(rows, cols):
    return pl.BlockSpec((rows, cols), lambda p, i: (0, 0))


_tcb = pl.pallas_call(
    _tcb_body,
    grid=(2, 7),
    in_specs=[_pblk2_spec(), _pblk2_spec(), _pblk28_spec(), _pblk28_spec(),
              _cst2(8, 128), _cst2(128, HALF), _cst2(HALF, 128), _cst2(8, HALF),
              _cst2(128, 128), _cst2(128, 128), _cst2(128, 128),
              _cst2(128, 128)],
    out_specs=[_pblk2_spec(), _pblk2_spec()],
    out_shape=[jax.ShapeDtypeStruct((PK, 128), jnp.float32)] * 2,
    scratch_shapes=[pltpu.VMEM((8, 128), jnp.float32)],
)

_tcc = pl.pallas_call(
    _tcc_body,
    grid=(7,),
    in_specs=[_pblk_spec(), _pblk_spec(), _pblk8_spec(), _pblk8_spec(),
              _cst(8, 128), _cst(128, HALF), _cst(HALF, 128), _cst(8, HALF)],
    out_specs=_cst(8, EMB),
    out_shape=jax.ShapeDtypeStruct((8, EMB), jnp.float32),
    scratch_shapes=[pltpu.VMEM((8, 128), jnp.float32)],
)


def kernel(edge_index, x, src_nodes, dst_nodes, user_emb, book_emb,
           W1, b1, g1, be1, W2, b2, g2, be2):
    f32 = jnp.float32
    zrows = jnp.zeros((NROWS - NN, HALF), f32)
    ha_p = jnp.concatenate(
        [user_emb[:, :HALF], book_emb[:, :HALF], zrows], 0).reshape(PK, 128)
    hb_p = jnp.concatenate(
        [user_emb[:, HALF:], book_emb[:, HALF:], zrows], 0).reshape(PK, 128)
    pad = EPAD - E
    srcp = jnp.concatenate([edge_index[0], jnp.zeros((pad,), jnp.int32)])
    dstp = jnp.concatenate(
        [edge_index[1], jnp.full((pad,), NN, jnp.int32)]).reshape(
            EPAD // 128, 128)

    i128 = jnp.arange(128)
    rmat = (i128[None, :] // HALF == jnp.arange(8)[:, None]).astype(f32)
    fmat = ((i128 % HALF)[:, None] == jnp.arange(HALF)[None, :]).astype(f32)
    ftmat = fmat.T

    eye8 = jnp.eye(8, dtype=f32)

    def bd(w, si, so):
        return jnp.kron(
            eye8, w[HALF * si:HALF * (si + 1), HALF * so:HALF * (so + 1)])

    def bnmat(b, g, be):
        return jnp.concatenate(
            [b.reshape(2, HALF), g.reshape(2, HALF), be.reshape(2, HALF),
             jnp.zeros((2, HALF), f32)], 0)

    d0, d1 = _deg_kernel(dstp)
    dp0 = d0.reshape(PK, 8)
    dp1 = d1.reshape(PK, 8)

    q1a, q1b, dv = _tca(ha_p, hb_p, dp0, dp1, rmat,
                        bd(W1, 0, 0), bd(W1, 0, 1), bd(W1, 1, 0), bd(W1, 1, 1))
    s1a, s1b = _mp_kernel(q1a.reshape(NROWS, HALF), q1b.reshape(NROWS, HALF),
                          srcp, dstp)
    q2a, q2b = _tcb(s1a.reshape(PK, 128), s1b.reshape(PK, 128), dp0, dp1,
                    rmat, fmat, ftmat, bnmat(b1, g1, be1),
                    bd(W2, 0, 0), bd(W2, 0, 1), bd(W2, 1, 0), bd(W2, 1, 1))
    s2a, s2b = _mp_kernel(q2a.reshape(NROWS, HALF), q2b.reshape(NROWS, HALF),
                          srcp, dstp)
    ac2 = _tcc(s2a.reshape(PK, 128), s2b.reshape(PK, 128), dp0, dp1,
               rmat, fmat, ftmat, bnmat(b2, g2, be2))
    ratings = _gd_kernel(s2a, s2b, dv.reshape(NROWS, HALF), ac2,
                         src_nodes.reshape(128, 128),
                         dst_nodes.reshape(128, 128))
    return ratings


# speedup vs baseline: 51.4669x; 1.1399x over previous
import functools

import jax
import jax.numpy as jnp
from jax import lax
from jax.experimental import pallas as pl
from jax.experimental.pallas import tpu as pltpu
from jax.experimental.pallas import tpu_sc as plsc

NN = 100000
NROWS = 100352
PK = NROWS // 8
PBLK = PK // 7
EMB = 32
HALF = 16
E = 1600000
NC, NS = 2, 16

CHUNK = 392
EPT = 100352
NCH = EPT // (4 * CHUNK)
EPAD = EPT * NS
ROWS_PT = NROWS // NS
ACC_ROWS = NROWS

DCHUNK = 1024
KJD = DCHUNK // 128
EPT_D = EPAD // (NC * NS)
NCH_D = EPT_D // DCHUNK
DEG_ROWS = NROWS
DROWS_PT = DEG_ROWS // NS

NP = 16384
PPT = NP // (NC * NS)
PJ = PPT // 128

_MESH = plsc.VectorSubcoreMesh(
    core_axis_name="c", subcore_axis_name="s", num_cores=NC, num_subcores=NS)
_SC_PARAMS = pltpu.CompilerParams(
    use_tc_tiling_on_sc=False, needs_layout_passes=False)


@functools.partial(
    pl.kernel,
    out_type=(jax.ShapeDtypeStruct((DEG_ROWS,), jnp.float32),
              jax.ShapeDtypeStruct((DEG_ROWS,), jnp.float32)),
    mesh=_MESH,
    compiler_params=_SC_PARAMS,
    scratch_types=[
        pltpu.VMEM_SHARED((DEG_ROWS,), jnp.float32),
        pltpu.VMEM((KJD, 128), jnp.int32),
        pltpu.VMEM((DROWS_PT,), jnp.float32),
        pltpu.VMEM((128,), jnp.float32),
        pltpu.SemaphoreType.DMA,
        pltpu.SemaphoreType.DMA,
    ],
)
def _deg_kernel(dst_hbm, d0_hbm, d1_hbm, dacc, idx_d, zbuf, ones_v, sem,
                ssem):
    cid = lax.axis_index("c")
    sid = lax.axis_index("s")
    r0 = sid * DROWS_PT

    def zb(i, carry):
        zbuf[pl.ds(i * 16, 16)] = jnp.zeros((16,), jnp.float32)
        return carry

    lax.fori_loop(0, DROWS_PT // 16, zb, 0)

    def ob(i, carry):
        ones_v[pl.ds(i * 16, 16)] = jnp.ones((16,), jnp.float32)
        return carry

    lax.fori_loop(0, 8, ob, 0)
    pltpu.sync_copy(zbuf, dacc.at[pl.ds(r0, DROWS_PT)])
    plsc.subcore_barrier()
    wid = cid * NS + sid
    base128 = wid * (EPT_D // 128)

    def body(ch, carry):
        off = base128 + ch * KJD
        pltpu.async_copy(dst_hbm.at[pl.ds(off, KJD)], idx_d, sem).wait()
        for j in range(KJD):
            pltpu.async_copy(ones_v, dacc.at[idx_d.at[j]], ssem, add=True)
        pltpu.make_async_copy(d0_hbm.at[pl.ds(0, DCHUNK)],
                              zbuf.at[pl.ds(0, DCHUNK)], ssem).wait()
        return carry

    lax.fori_loop(0, NCH_D, body, 0)
    plsc.subcore_barrier()
    pltpu.sync_copy(dacc.at[pl.ds(r0, DROWS_PT)], zbuf)

    @pl.when(cid == 0)
    def _():
        pltpu.sync_copy(zbuf, d0_hbm.at[pl.ds(r0, DROWS_PT)])

    @pl.when(cid == 1)
    def _():
        pltpu.sync_copy(zbuf, d1_hbm.at[pl.ds(r0, DROWS_PT)])


@functools.partial(
    pl.kernel,
    out_type=(jax.ShapeDtypeStruct((NROWS, HALF), jnp.float32),
              jax.ShapeDtypeStruct((NROWS, HALF), jnp.float32)),
    mesh=_MESH,
    compiler_params=_SC_PARAMS,
    scratch_types=[
        pltpu.VMEM_SHARED((ACC_ROWS, HALF), jnp.float32),
        pltpu.VMEM((2 * CHUNK,), jnp.int32),
        pltpu.VMEM((2 * CHUNK,), jnp.int32),
        pltpu.VMEM((2 * CHUNK,), jnp.int32),
        pltpu.VMEM((2 * CHUNK,), jnp.int32),
        pltpu.VMEM((CHUNK, HALF), jnp.float32),
        pltpu.VMEM((CHUNK, HALF), jnp.float32),
        pltpu.VMEM((CHUNK, HALF), jnp.float32),
        pltpu.VMEM((CHUNK, HALF), jnp.float32),
        pltpu.SemaphoreType.DMA,
        pltpu.SemaphoreType.DMA,
        pltpu.SemaphoreType.DMA,
        pltpu.SemaphoreType.DMA,
        pltpu.SemaphoreType.DMA,
        pltpu.SemaphoreType.DMA,
        pltpu.SemaphoreType.DMA,
    ],
)
def _mp_kernel(q0_hbm, q1_hbm, src_hbm, dst_hbm, a0_hbm, a1_hbm,
               acc, ixsx, ixsy, ixdx, ixdy, ra, rb, rc, rd,
               gsa, gsb, gsc, gsd, ssx, ssy, isem):
    cid = lax.axis_index("c")
    sid = lax.axis_index("s")
    r0 = sid * ROWS_PT
    chunks = (CHUNK,) * (ROWS_PT // CHUNK)

    def stage(hbm_ref, to_spmem):
        off = 0
        for cnt in chunks:
            hslice = hbm_ref.at[pl.ds(r0 + off, cnt)]
            vslice = ra.at[pl.ds(0, cnt)]
            aslice = acc.at[pl.ds(r0 + off, cnt)]
            if to_spmem:
                pltpu.sync_copy(hslice, vslice)
                pltpu.sync_copy(vslice, aslice)
            else:
                pltpu.sync_copy(aslice, vslice)
                pltpu.sync_copy(vslice, hslice)
            off += cnt

    @pl.when(cid == 0)
    def _():
        stage(q0_hbm, True)

    @pl.when(cid == 1)
    def _():
        stage(q1_hbm, True)

    plsc.subcore_barrier()
    base = sid * EPT

    def run(q_hbm):
        def fire_gather(rows_buf, sem, ixs, slot):
            pltpu.async_copy(q_hbm.at[ixs.at[pl.ds(slot * CHUNK, CHUNK)]],
                             rows_buf, sem)

        def drain(sem):
            pltpu.make_async_copy(q_hbm.at[pl.ds(0, CHUNK)], ra, sem).wait()

        def fire_scatter(rows_buf, sem, ixd, slot):
            pltpu.async_copy(rows_buf,
                             acc.at[ixd.at[pl.ds(slot * CHUNK, CHUNK)]],
                             sem, add=True)

        def copy_idx(off, ixs, ixd):
            c1 = pltpu.async_copy(
                src_hbm.at[pl.ds(off, 2 * CHUNK)], ixs, isem)
            c2 = pltpu.async_copy(
                dst_hbm.at[pl.ds(off, 2 * CHUNK)], ixd, isem)
            c1.wait()
            c2.wait()

        def body(i, carry):
            off = base * 1 + i * (4 * CHUNK)
            copy_idx(off, ixsx, ixdx)
            fire_gather(ra, gsa, ixsx, 0)
            fire_gather(rb, gsb, ixsx, 1)

            @pl.when(i > 0)
            def _():
                drain(ssy)
                drain(ssy)

            copy_idx(off + 2 * CHUNK, ixsy, ixdy)
            drain(gsa)
            fire_scatter(ra, ssx, ixdx, 0)
            drain(gsb)
            fire_scatter(rb, ssx, ixdx, 1)
            fire_gather(rc, gsc, ixsy, 0)
            fire_gather(rd, gsd, ixsy, 1)
            drain(ssx)
            drain(ssx)
            drain(gsc)
            fire_scatter(rc, ssy, ixdy, 0)
            drain(gsd)
            fire_scatter(rd, ssy, ixdy, 1)
            return carry

        lax.fori_loop(0, NCH, body, 0)
        drain(ssy)
        drain(ssy)

    @pl.when(cid == 0)
    def _():
        run(q0_hbm)

    @pl.when(cid == 1)
    def _():
        run(q1_hbm)

    plsc.subcore_barrier()

    @pl.when(cid == 0)
    def _():
        stage(a0_hbm, False)

    @pl.when(cid == 1)
    def _():
        stage(a1_hbm, False)


@functools.partial(
    pl.kernel,
    out_type=jax.ShapeDtypeStruct((NP,), jnp.float32),
    mesh=_MESH,
    compiler_params=_SC_PARAMS,
    scratch_types=[
        pltpu.VMEM((PJ, 128), jnp.int32),
        pltpu.VMEM((PJ, 128), jnp.int32),
        pltpu.VMEM((128, HALF), jnp.float32),
        pltpu.VMEM((128, HALF), jnp.float32),
        pltpu.VMEM((128, HALF), jnp.float32),
        pltpu.VMEM((128, HALF), jnp.float32),
        pltpu.VMEM((128, HALF), jnp.float32),
        pltpu.VMEM((128, HALF), jnp.float32),
        pltpu.VMEM((8, EMB), jnp.float32),
        pltpu.VMEM((PPT,), jnp.float32),
        pltpu.SemaphoreType.DMA,
    ],
)
def _gd_kernel(sa_hbm, sb_hbm, dv_hbm, ac_hbm, srcn_hbm, dstn_hbm, r_hbm,
               idx_a, idx_b, bufsa, bufsb, bufda, bufdb, dvs, dvd,
               ac_v, res, sem):
    cid = lax.axis_index("c")
    sid = lax.axis_index("s")
    wid = cid * NS + sid
    pltpu.sync_copy(ac_hbm, ac_v)
    pltpu.sync_copy(srcn_hbm.at[pl.ds(wid * PJ, PJ)], idx_a)
    pltpu.sync_copy(dstn_hbm.at[pl.ds(wid * PJ, PJ)], idx_b)
    lane = lax.iota(jnp.int32, 16)
    zero16 = jnp.zeros((16,), jnp.int32)
    one16 = zero16 + 1

    def jbody(j, carry):
        cps = [
            pltpu.async_copy(sa_hbm.at[idx_a.at[j]], bufsa, sem),
            pltpu.async_copy(sb_hbm.at[idx_a.at[j]], bufsb, sem),
            pltpu.async_copy(dv_hbm.at[idx_a.at[j]], dvs, sem),
            pltpu.async_copy(sa_hbm.at[idx_b.at[j]], bufda, sem),
            pltpu.async_copy(sb_hbm.at[idx_b.at[j]], bufdb, sem),
            pltpu.async_copy(dv_hbm.at[idx_b.at[j]], dvd, sem),
        ]
        for cp in cps:
            cp.wait()

        def gbody(g, carry2):
            rowi = g * 16 + lane
            acc16 = jnp.zeros((16,), jnp.float32)
            for h, (bs, bd) in enumerate(((bufsa, bufda), (bufsb, bufdb))):
                def cbody(cc, acc_in, bs=bs, bd=bd, h=h):
                    col = zero16 + cc
                    a_c = plsc.load_gather(ac_v, [zero16, col + h * HALF])
                    c_c = plsc.load_gather(ac_v, [one16, col + h * HALF])
                    va = plsc.load_gather(bs, [rowi, col])
                    vb = plsc.load_gather(bd, [rowi, col])
                    da = plsc.load_gather(dvs, [rowi, col])
                    db = plsc.load_gather(dvd, [rowi, col])
                    ya = jnp.maximum(a_c * (va * da) + c_c, 0.0)
                    yb = jnp.maximum(a_c * (vb * db) + c_c, 0.0)
                    return acc_in + ya * yb

                acc16 = lax.fori_loop(0, HALF, cbody, acc16)
            res[pl.ds(j * 128 + g * 16, 16)] = acc16
            return carry2

        lax.fori_loop(0, 8, gbody, 0)
        return carry

    lax.fori_loop(0, PJ, jbody, 0)
    pltpu.sync_copy(res, r_hbm.at[pl.ds(wid * PPT, PPT)])


NN_F = float(NN)


def _tca_body(ha_ref, hb_ref, dp0_ref, dp1_ref, r_ref,
              waa_ref, wab_ref, wba_ref, wbb_ref,
              qa_ref, qb_ref, dv_ref):
    deg = dp0_ref[...] + dp1_ref[...] + 1.0
    dinvp = jnp.dot(lax.rsqrt(deg), r_ref[...],
                    preferred_element_type=jnp.float32,
                    precision=lax.Precision.HIGHEST)
    dv_ref[...] = dinvp
    ha = ha_ref[...]
    hb = hb_ref[...]
    dot = functools.partial(jnp.dot, preferred_element_type=jnp.float32)
    qa_ref[...] = (dot(ha, waa_ref[...]) + dot(hb, wba_ref[...])) * dinvp
    qb_ref[...] = (dot(ha, wab_ref[...]) + dot(hb, wbb_ref[...])) * dinvp


def _tcb_body(sa_ref, sb_ref, dp0_ref, dp1_ref, r_ref, f_ref, ft_ref,
              bn_ref, waa_ref, wab_ref, wba_ref, wbb_ref,
              qa_ref, qb_ref, accum):
    p = pl.program_id(0)
    i = pl.program_id(1)
    dot = functools.partial(jnp.dot, preferred_element_type=jnp.float32)
    dotx = functools.partial(jnp.dot, preferred_element_type=jnp.float32,
                             precision=lax.Precision.HIGHEST)
    deg = dp0_ref[...] + dp1_ref[...] + 1.0
    dinvp = dotx(lax.rsqrt(deg), r_ref[...])
    bap = dotx(bn_ref[0:1, :], ft_ref[...])
    bbp = dotx(bn_ref[1:2, :], ft_ref[...])
    outa = sa_ref[...] * dinvp + bap
    outb = sb_ref[...] * dinvp + bbp

    @pl.when(p == 0)
    def _():
        @pl.when(i == 0)
        def _():
            accum[...] = jnp.zeros_like(accum)

        rowid = lax.broadcasted_iota(jnp.int32, (PBLK, 128), 0) + i * PBLK
        m = rowid < (NN // 8)
        oa = jnp.where(m, outa, 0.0)
        ob = jnp.where(m, outb, 0.0)
        accum[0:1, :] += jnp.sum(oa, axis=0, keepdims=True)
        accum[1:2, :] += jnp.sum(oa * oa, axis=0, keepdims=True)
        accum[2:3, :] += jnp.sum(ob, axis=0, keepdims=True)
        accum[3:4, :] += jnp.sum(ob * ob, axis=0, keepdims=True)

        @pl.when(i == 6)
        def _():
            mu_a = dotx(accum[0:1, :], f_ref[...]) / NN_F
            var_a = dotx(accum[1:2, :], f_ref[...]) / NN_F - mu_a * mu_a
            mu_b = dotx(accum[2:3, :], f_ref[...]) / NN_F
            var_b = dotx(accum[3:4, :], f_ref[...]) / NN_F - mu_b * mu_b
            a16a = bn_ref[2:3, :] * lax.rsqrt(var_a + 1e-5)
            c16a = bn_ref[4:5, :] - a16a * mu_a
            a16b = bn_ref[3:4, :] * lax.rsqrt(var_b + 1e-5)
            c16b = bn_ref[5:6, :] - a16b * mu_b
            accum[4:5, :] = dotx(a16a, ft_ref[...])
            accum[5:6, :] = dotx(c16a, ft_ref[...])
            accum[6:7, :] = dotx(a16b, ft_ref[...])
            accum[7:8, :] = dotx(c16b, ft_ref[...])

    @pl.when(p == 1)
    def _():
        h1a = jnp.maximum(accum[4:5, :] * outa + accum[5:6, :], 0.0)
        h1b = jnp.maximum(accum[6:7, :] * outb + accum[7:8, :], 0.0)
        qa_ref[...] = (dot(h1a, waa_ref[...]) + dot(h1b, wba_ref[...])) * dinvp
        qb_ref[...] = (dot(h1a, wab_ref[...]) + dot(h1b, wbb_ref[...])) * dinvp


def _tcc_body(sa_ref, sb_ref, dp0_ref, dp1_ref, r_ref, f_ref, ft_ref,
              bn_ref, ac_ref, accum):
    i = pl.program_id(0)
    dot = functools.partial(jnp.dot, preferred_element_type=jnp.float32)
    dotx = functools.partial(jnp.dot, preferred_element_type=jnp.float32,
                             precision=lax.Precision.HIGHEST)
    deg = dp0_ref[...] + dp1_ref[...] + 1.0
    dinvp = dotx(lax.rsqrt(deg), r_ref[...])
    bap = dotx(bn_ref[0:1, :], ft_ref[...])
    bbp = dotx(bn_ref[1:2, :], ft_ref[...])
    outa = sa_ref[...] * dinvp + bap
    outb = sb_ref[...] * dinvp + bbp

    @pl.when(i == 0)
    def _():
        accum[...] = jnp.zeros_like(accum)

    rowid = lax.broadcasted_iota(jnp.int32, (PBLK, 128), 0) + i * PBLK
    m = rowid < (NN // 8)
    oa = jnp.where(m, outa, 0.0)
    ob = jnp.where(m, outb, 0.0)
    accum[0:1, :] += jnp.sum(oa, axis=0, keepdims=True)
    accum[1:2, :] += jnp.sum(oa * oa, axis=0, keepdims=True)
    accum[2:3, :] += jnp.sum(ob, axis=0, keepdims=True)
    accum[3:4, :] += jnp.sum(ob * ob, axis=0, keepdims=True)

    @pl.when(i == 6)
    def _():
        mu_a = dotx(accum[0:1, :], f_ref[...]) / NN_F
        var_a = dotx(accum[1:2, :], f_ref[...]) / NN_F - mu_a * mu_a
        mu_b = dotx(accum[2:3, :], f_ref[...]) / NN_F
        var_b = dotx(accum[3:4, :], f_ref[...]) / NN_F - mu_b * mu_b
        a16a = bn_ref[2:3, :] * lax.rsqrt(var_a + 1e-5)
        c16a = bn_ref[4:5, :] - a16a * mu_a
        a16b = bn_ref[3:4, :] * lax.rsqrt(var_b + 1e-5)
        c16b = bn_ref[5:6, :] - a16b * mu_b
        ca = c16a + a16a * bn_ref[0:1, :]
        cb = c16b + a16b * bn_ref[1:2, :]
        ac_ref[0:1, :] = jnp.concatenate([a16a, a16b], axis=1)
        ac_ref[1:2, :] = jnp.concatenate([ca, cb], axis=1)
        ac_ref[2:8, :] = jnp.zeros((6, EMB), jnp.float32)


def _pblk_spec():
    return pl.BlockSpec((PBLK, 128), lambda i: (i, 0))


def _pblk8_spec():
    return pl.BlockSpec((PBLK, 8), lambda i: (i, 0))


def _cst(rows, cols):
    return pl.BlockSpec((rows, cols), lambda i: (0, 0))


_tca = pl.pallas_call(
    _tca_body,
    grid=(7,),
    in_specs=[_pblk_spec(), _pblk_spec(), _pblk8_spec(), _pblk8_spec(),
              _cst(8, 128), _cst(128, 128), _cst(128, 128), _cst(128, 128),
              _cst(128, 128)],
    out_specs=[_pblk_spec(), _pblk_spec(), _pblk_spec()],
    out_shape=[jax.ShapeDtypeStruct((PK, 128), jnp.float32)] * 3,
)


def _pblk2_spec():
    return pl.BlockSpec((PBLK, 128), lambda p, i: (i, 0))


def _pblk28_spec():
    return pl.BlockSpec((PBLK, 8), lambda p, i: (i, 0))


def _cst2(rows, cols):
    return pl.BlockSpec((rows, cols), lambda p, i: (0, 0))


_tcb = pl.pallas_call(
    _tcb_body,
    grid=(2, 7),
    in_specs=[_pblk2_spec(), _pblk2_spec(), _pblk28_spec(), _pblk28_spec(),
              _cst2(8, 128), _cst2(128, HALF), _cst2(HALF, 128), _cst2(8, HALF),
              _cst2(128, 128), _cst2(128, 128), _cst2(128, 128),
              _cst2(128, 128)],
    out_specs=[_pblk2_spec(), _pblk2_spec()],
    out_shape=[jax.ShapeDtypeStruct((PK, 128), jnp.float32)] * 2,
    scratch_shapes=[pltpu.VMEM((8, 128), jnp.float32)],
)

_tcc = pl.pallas_call(
    _tcc_body,
    grid=(7,),
    in_specs=[_pblk_spec(), _pblk_spec(), _pblk8_spec(), _pblk8_spec(),
              _cst(8, 128), _cst(128, HALF), _cst(HALF, 128), _cst(8, HALF)],
    out_specs=_cst(8, EMB),
    out_shape=jax.ShapeDtypeStruct((8, EMB), jnp.float32),
    scratch_shapes=[pltpu.VMEM((8, 128), jnp.float32)],
)


def kernel(edge_index, x, src_nodes, dst_nodes, user_emb, book_emb,
           W1, b1, g1, be1, W2, b2, g2, be2):
    f32 = jnp.float32
    zrows = jnp.zeros((NROWS - NN, HALF), f32)
    ha_p = jnp.concatenate(
        [user_emb[:, :HALF], book_emb[:, :HALF], zrows], 0).reshape(PK, 128)
    hb_p = jnp.concatenate(
        [user_emb[:, HALF:], book_emb[:, HALF:], zrows], 0).reshape(PK, 128)
    pad = EPAD - E
    srcp = jnp.concatenate([edge_index[0], jnp.zeros((pad,), jnp.int32)])
    dst1 = jnp.concatenate([edge_index[1], jnp.full((pad,), NN, jnp.int32)])
    dstp = dst1.reshape(EPAD // 128, 128)

    i128 = jnp.arange(128)
    rmat = (i128[None, :] // HALF == jnp.arange(8)[:, None]).astype(f32)
    fmat = ((i128 % HALF)[:, None] == jnp.arange(HALF)[None, :]).astype(f32)
    ftmat = fmat.T

    eye8 = jnp.eye(8, dtype=f32)

    def bd(w, si, so):
        return jnp.kron(
            eye8, w[HALF * si:HALF * (si + 1), HALF * so:HALF * (so + 1)])

    def bnmat(b, g, be):
        return jnp.concatenate(
            [b.reshape(2, HALF), g.reshape(2, HALF), be.reshape(2, HALF),
             jnp.zeros((2, HALF), f32)], 0)

    d0, d1 = _deg_kernel(dstp)
    dp0 = d0.reshape(PK, 8)
    dp1 = d1.reshape(PK, 8)

    q1a, q1b, dv = _tca(ha_p, hb_p, dp0, dp1, rmat,
                        bd(W1, 0, 0), bd(W1, 0, 1), bd(W1, 1, 0), bd(W1, 1, 1))
    s1a, s1b = _mp_kernel(q1a.reshape(NROWS, HALF), q1b.reshape(NROWS, HALF),
                          srcp, dst1)
    q2a, q2b = _tcb(s1a.reshape(PK, 128), s1b.reshape(PK, 128), dp0, dp1,
                    rmat, fmat, ftmat, bnmat(b1, g1, be1),
                    bd(W2, 0, 0), bd(W2, 0, 1), bd(W2, 1, 0), bd(W2, 1, 1))
    s2a, s2b = _mp_kernel(q2a.reshape(NROWS, HALF), q2b.reshape(NROWS, HALF),
                          srcp, dst1)
    ac2 = _tcc(s2a.reshape(PK, 128), s2b.reshape(PK, 128), dp0, dp1,
               rmat, fmat, ftmat, bnmat(b2, g2, be2))
    ratings = _gd_kernel(s2a, s2b, dv.reshape(NROWS, HALF), ac2,
                         src_nodes.reshape(128, 128),
                         dst_nodes.reshape(128, 128))
    return ratings
